# baseline (speedup 1.0000x reference)
import functools

import jax
import jax.numpy as jnp
from jax import lax
from jax.experimental import pallas as pl
from jax.experimental.pallas import tpu as pltpu
from jax.experimental.pallas import tpu_sc as plsc

_N = 10000
_E = 160000
_D_IN = 128
_D_EDGE = 16
_HID = 64
_OUT = 64
_BN_EPS = 1e-5

_CHUNK = 128
_NCHUNKS = _E // _CHUNK
_NW = 32
_CHUNKS_PER_W = -(-_NCHUNKS // _NW)
_NPAD = 10240
_ROWS_PER_SUB = _NPAD // 16

_BE = 2000
_NBLK = _E // _BE


def _sc_mesh():
    return plsc.VectorSubcoreMesh(core_axis_name="c", subcore_axis_name="s")


def _worker_id():
    return lax.axis_index("s") * 2 + lax.axis_index("c")


_IO_ROWS = 640
_IO_SUB = _IO_ROWS // _CHUNK
_N_IO = _E // _IO_ROWS
_K_PER_W = -(-_N_IO // _NW)


def _make_gather_body(njobs, width, dtype):
    def body(*refs):
        idxs = refs[:njobs]
        table = refs[njobs]
        outs = refs[njobs + 1:2 * njobs + 1]
        sc = refs[2 * njobs + 1:]
        wid = _worker_id()

        def guard(k):
            return (k >= 0) & ((wid + k * _NW) < _N_IO)

        def addr(k):
            c = wid + k * _NW
            return (pl.multiple_of(c * _IO_SUB, 8),
                    pl.multiple_of(c * _IO_ROWS, 8))

        def fire(k, b):
            @pl.when(guard(k))
            def _():
                i0, _r0 = addr(k)
                for j in range(njobs):
                    idx_v = sc[8 * j + b]
                    rows_v = sc[8 * j + 2 + b]
                    gsem = sc[8 * j + 4 + b]
                    pltpu.sync_copy(idxs[j].at[pl.ds(i0, _IO_SUB)], idx_v)
                    for s in range(_IO_SUB):
                        pltpu.async_copy(
                            table.at[idx_v.at[s]],
                            rows_v.at[pl.ds(s * _CHUNK, _CHUNK)], gsem)

        def drain_gather_fire_wb(k, b):
            @pl.when(guard(k))
            def _():
                _i0, r0 = addr(k)
                for j in range(njobs):
                    idx_v = sc[8 * j + b]
                    rows_v = sc[8 * j + 2 + b]
                    gsem = sc[8 * j + 4 + b]
                    wsem = sc[8 * j + 6 + b]
                    for s in range(_IO_SUB):
                        pltpu.make_async_copy(
                            table.at[idx_v.at[s]],
                            rows_v.at[pl.ds(s * _CHUNK, _CHUNK)], gsem).wait()
                    pltpu.async_copy(rows_v, outs[j].at[pl.ds(r0, _IO_ROWS)],
                                     wsem)

        def drain_wb(k, b):
            @pl.when(guard(k))
            def _():
                _i0, r0 = addr(k)
                for j in range(njobs):
                    rows_v = sc[8 * j + 2 + b]
                    wsem = sc[8 * j + 6 + b]
                    pltpu.make_async_copy(
                        rows_v, outs[j].at[pl.ds(r0, _IO_ROWS)], wsem).wait()

        def step(i, carry):
            kk = 2 * i
            drain_wb(kk - 2, 0)
            fire(kk, 0)
            drain_wb(kk - 1, 1)
            fire(kk + 1, 1)
            drain_gather_fire_wb(kk, 0)
            drain_gather_fire_wb(kk + 1, 1)
            return carry

        lax.fori_loop(0, _K_PER_W // 2, step, 0)
        drain_wb(_K_PER_W - 2, 0)
        drain_wb(_K_PER_W - 1, 1)

    return body


def _sc_gather(table, idxs, width, dtype):
    njobs = len(idxs)
    scratch = []
    for _ in range(njobs):
        scratch += [pltpu.VMEM((_IO_SUB, _CHUNK), jnp.int32)] * 2
        scratch += [pltpu.VMEM((_IO_ROWS, width), dtype)] * 2
        scratch += [pltpu.SemaphoreType.DMA] * 4
    out = pl.kernel(
        _make_gather_body(njobs, width, dtype),
        out_type=[jax.ShapeDtypeStruct((_E, width), dtype)] * njobs,
        mesh=_sc_mesh(),
        scratch_types=scratch,
        compiler_params=pltpu.CompilerParams(use_tc_tiling_on_sc=False),
    )(*idxs, table)
    return out


def _scatter_body(dst_hbm, msg_hbm, zeros_hbm, out_hbm,
                  idx0, idx1, msg0, msg1, msem0, msem1, asem0, asem1, agg_sh):
    cid = lax.axis_index("c")
    sid = lax.axis_index("s")
    wid = _worker_id()
    idx_b = (idx0, idx1)
    msg_b = (msg0, msg1)
    msem_b = (msem0, msem1)
    asem_b = (asem0, asem1)

    row0 = pl.multiple_of(sid * _ROWS_PER_SUB, 8)
    pltpu.sync_copy(zeros_hbm, agg_sh.at[pl.ds(row0, _ROWS_PER_SUB)])
    plsc.subcore_barrier()

    def guard(k):
        return (k >= 0) & ((wid + k * _NW) < _N_IO)

    def addr(k):
        c = wid + k * _NW
        return (pl.multiple_of(c * _IO_SUB, 8),
                pl.multiple_of(c * _IO_ROWS, 8))

    def fire(k, b):
        @pl.when(guard(k))
        def _():
            i0, r0 = addr(k)
            pltpu.sync_copy(dst_hbm.at[pl.ds(i0, _IO_SUB)], idx_b[b])
            pltpu.async_copy(msg_hbm.at[pl.ds(r0, _IO_ROWS)], msg_b[b],
                             msem_b[b])

    def drain_msg_fire_add(k, b):
        @pl.when(guard(k))
        def _():
            _i0, r0 = addr(k)
            pltpu.make_async_copy(msg_hbm.at[pl.ds(r0, _IO_ROWS)], msg_b[b],
                                  msem_b[b]).wait()
            for s in range(_IO_SUB):
                pltpu.async_copy(msg_b[b].at[pl.ds(s * _CHUNK, _CHUNK)],
                                 agg_sh.at[idx_b[b].at[s]], asem_b[b],
                                 add=True)

    def drain_add(k, b):
        @pl.when(guard(k))
        def _():
            for s in range(_IO_SUB):
                pltpu.make_async_copy(msg_b[b].at[pl.ds(s * _CHUNK, _CHUNK)],
                                      agg_sh.at[idx_b[b].at[s]],
                                      asem_b[b]).wait()

    def step(i, carry):
        kk = 2 * i
        drain_add(kk - 2, 0)
        fire(kk, 0)
        drain_add(kk - 1, 1)
        fire(kk + 1, 1)
        drain_msg_fire_add(kk, 0)
        drain_msg_fire_add(kk + 1, 1)
        return carry

    lax.fori_loop(0, _K_PER_W // 2, step, 0)
    drain_add(_K_PER_W - 2, 0)
    drain_add(_K_PER_W - 1, 1)
    plsc.subcore_barrier()
    out0 = pl.multiple_of(cid * _NPAD + row0, 8)
    pltpu.sync_copy(agg_sh.at[pl.ds(row0, _ROWS_PER_SUB)],
                    out_hbm.at[pl.ds(out0, _ROWS_PER_SUB)])


def _sc_scatter_add(msg, dst2d):
    zeros = jnp.zeros((_ROWS_PER_SUB, _HID), jnp.float32)
    return pl.kernel(
        _scatter_body,
        out_type=jax.ShapeDtypeStruct((2 * _NPAD, _HID), jnp.float32),
        mesh=_sc_mesh(),
        scratch_types=[
            pltpu.VMEM((_IO_SUB, _CHUNK), jnp.int32),
            pltpu.VMEM((_IO_SUB, _CHUNK), jnp.int32),
            pltpu.VMEM((_IO_ROWS, _HID), jnp.float32),
            pltpu.VMEM((_IO_ROWS, _HID), jnp.float32),
            pltpu.SemaphoreType.DMA,
            pltpu.SemaphoreType.DMA,
            pltpu.SemaphoreType.DMA,
            pltpu.SemaphoreType.DMA,
            pltpu.VMEM_SHARED((_NPAD, _HID), jnp.float32),
        ],
        compiler_params=pltpu.CompilerParams(use_tc_tiling_on_sc=False),
    )(dst2d, msg, zeros)


def _msg_body(hs_ref, ef_ref, a_ref, b_ref, eps_ref, out_ref):
    hs = hs_ref[...]
    acc = jnp.dot(hs, b_ref[...], preferred_element_type=jnp.float32)
    ef = ef_ref[...]
    for f in range(_D_EDGE):
        acc += ef[:, f:f + 1] * jnp.dot(hs, a_ref[f],
                                        preferred_element_type=jnp.float32)
    out_ref[...] = (1.0 + eps_ref[0, 0]) * acc


def _tc_msg(h_src, efeat, a, bmat, eps):
    return pl.pallas_call(
        _msg_body,
        grid=(_NBLK,),
        in_specs=[
            pl.BlockSpec((_BE, _D_IN), lambda i: (i, 0)),
            pl.BlockSpec((_BE, _D_EDGE), lambda i: (i, 0)),
            pl.BlockSpec((_D_EDGE, _D_IN, _HID), lambda i: (0, 0, 0)),
            pl.BlockSpec((_D_IN, _HID), lambda i: (0, 0)),
            pl.BlockSpec((1, 1), lambda i: (0, 0)),
        ],
        out_specs=pl.BlockSpec((_BE, _HID), lambda i: (i, 0)),
        out_shape=jax.ShapeDtypeStruct((_E, _HID), jnp.float32),
    )(h_src, efeat, a, bmat, eps)


def _bn_cols(x, g, b):
    mu = jnp.mean(x, axis=0, keepdims=True)
    xc = x - mu
    var = jnp.mean(xc * xc, axis=0, keepdims=True)
    return xc * lax.rsqrt(var + _BN_EPS) * g + b


def _leaky(x):
    return jnp.where(x >= 0, x, 0.01 * x)


def _node_body(agg_ref, w1_ref, b1_ref, g1_ref, be1_ref, w2_ref, b2_ref,
               g_ref, be_ref, wo1_ref, bo1_ref, pw_ref, pb_ref,
               v_ref, hp_ref):
    agg = agg_ref[0:_N, :] + agg_ref[_NPAD:_NPAD + _N, :]
    x = jnp.dot(agg, w1_ref[...], preferred_element_type=jnp.float32) + b1_ref[...]
    hr = jnp.maximum(_bn_cols(x, g1_ref[...], be1_ref[...]), 0.0)
    x2 = jnp.dot(hr, w2_ref[...], preferred_element_type=jnp.float32) + b2_ref[...]
    h1 = _leaky(_bn_cols(x2, g_ref[...], be_ref[...]))
    v_ref[...] = jnp.dot(h1, wo1_ref[...], preferred_element_type=jnp.float32) + bo1_ref[...]
    hp = jnp.dot(h1, pw_ref[...], preferred_element_type=jnp.float32) + pb_ref[...]
    hp_ref[...] = hp.astype(jnp.bfloat16)


def _tc_node(agg2, p):
    full = lambda s: pl.BlockSpec(s, lambda: tuple(0 for _ in s))
    return pl.pallas_call(
        _node_body,
        in_specs=[
            full((2 * _NPAD, _HID)),
            full((_HID, _HID)), full((1, _HID)), full((1, _HID)), full((1, _HID)),
            full((_HID, _HID)), full((1, _HID)), full((1, _HID)), full((1, _HID)),
            full((_HID, _OUT)), full((1, _OUT)),
            full((_HID, _D_EDGE)), full((1, _D_EDGE)),
        ],
        out_specs=[full((_N, _OUT)), full((_N, _D_EDGE))],
        out_shape=[
            jax.ShapeDtypeStruct((_N, _OUT), jnp.float32),
            jax.ShapeDtypeStruct((_N, _D_EDGE), jnp.bfloat16),
        ],
    )(agg2,
      p['nc_w1'], p['nc_b1'].reshape(1, -1), p['nc_g1'].reshape(1, -1),
      p['nc_be1'].reshape(1, -1),
      p['nc_w2'], p['nc_b2'].reshape(1, -1), p['nc_g'].reshape(1, -1),
      p['nc_be'].reshape(1, -1),
      p['wo1'], p['bo1'].reshape(1, -1),
      p['ec_pw'], p['ec_pb'].reshape(1, -1))


def _hein_body(ef_ref, hs_ref, hd_ref, eps_ref, out_ref, s1_ref, m1_ref):
    i = pl.program_id(0)
    he = ((1.0 + eps_ref[0, 0]) * ef_ref[...]
          + hs_ref[...].astype(jnp.float32) + hd_ref[...].astype(jnp.float32))
    out_ref[...] = he
    s = jnp.sum(he, axis=0, keepdims=True)
    m = lax.dot_general(he, he, (((0,), (0,)), ((), ())),
                        preferred_element_type=jnp.float32)

    @pl.when(i == 0)
    def _():
        s1_ref[...] = s
        m1_ref[...] = m

    @pl.when(i > 0)
    def _():
        s1_ref[...] += s
        m1_ref[...] += m


def _tc_hein(efeat, hp_src, hp_dst, eps):
    blk = lambda w: pl.BlockSpec((_BE, w), lambda i: (i, 0))
    return pl.pallas_call(
        _hein_body,
        grid=(_NBLK,),
        in_specs=[blk(_D_EDGE), blk(_D_EDGE), blk(_D_EDGE),
                  pl.BlockSpec((1, 1), lambda i: (0, 0))],
        out_specs=[
            pl.BlockSpec((_BE, _D_EDGE), lambda i: (i, 0)),
            pl.BlockSpec((1, _D_EDGE), lambda i: (0, 0)),
            pl.BlockSpec((_D_EDGE, _D_EDGE), lambda i: (0, 0)),
        ],
        out_shape=[
            jax.ShapeDtypeStruct((_E, _D_EDGE), jnp.float32),
            jax.ShapeDtypeStruct((1, _D_EDGE), jnp.float32),
            jax.ShapeDtypeStruct((_D_EDGE, _D_EDGE), jnp.float32),
        ],
    )(efeat, hp_src, hp_dst, eps)


def _bn_stats(s, m, w, b):
    mean_y = s / _E
    mw = jnp.dot(mean_y, w, preferred_element_type=jnp.float32)
    mu = mw + b
    diag = jnp.sum(w * jnp.dot(m, w, preferred_element_type=jnp.float32),
                   axis=0, keepdims=True)
    ex2 = diag / _E + 2.0 * b * mw + b * b
    return mu, ex2 - mu * mu


def _stats2_body(he_ref, s1_ref, m1_ref, w1_ref, b1_ref, g1_ref, be1_ref,
                 s2_ref, m2_ref):
    i = pl.program_id(0)
    w1 = w1_ref[...]
    b1 = b1_ref[...]
    mu1, var1 = _bn_stats(s1_ref[...], m1_ref[...], w1, b1)
    x = jnp.dot(he_ref[...], w1, preferred_element_type=jnp.float32) + b1
    xn = (x - mu1) * lax.rsqrt(var1 + _BN_EPS) * g1_ref[...] + be1_ref[...]
    hr = jnp.maximum(xn, 0.0)
    s = jnp.sum(hr, axis=0, keepdims=True)
    m = lax.dot_general(hr, hr, (((0,), (0,)), ((), ())),
                        preferred_element_type=jnp.float32)

    @pl.when(i == 0)
    def _():
        s2_ref[...] = s
        m2_ref[...] = m

    @pl.when(i > 0)
    def _():
        s2_ref[...] += s
        m2_ref[...] += m


def _tc_stats2(he_in, s1, m1, p):
    full = lambda s: pl.BlockSpec(s, lambda i: tuple(0 for _ in s))
    return pl.pallas_call(
        _stats2_body,
        grid=(_NBLK,),
        in_specs=[
            pl.BlockSpec((_BE, _D_EDGE), lambda i: (i, 0)),
            full((1, _D_EDGE)), full((_D_EDGE, _D_EDGE)),
            full((_D_EDGE, _HID)), full((1, _HID)), full((1, _HID)),
            full((1, _HID)),
        ],
        out_specs=[full((1, _HID)), full((_HID, _HID))],
        out_shape=[
            jax.ShapeDtypeStruct((1, _HID), jnp.float32),
            jax.ShapeDtypeStruct((_HID, _HID), jnp.float32),
        ],
    )(he_in, s1, m1, p['ec_w1'], p['ec_b1'].reshape(1, -1),
      p['ec_g1'].reshape(1, -1), p['ec_be1'].reshape(1, -1))


def _edge_out_body(he_ref, s1_ref, m1_ref, s2_ref, m2_ref, w1_ref, b1_ref,
                   g1_ref, be1_ref, w2_ref, b2_ref, g_ref, be_ref,
                   wo2_ref, bo2_ref, out_ref):
    w1 = w1_ref[...]
    b1 = b1_ref[...]
    w2 = w2_ref[...]
    b2 = b2_ref[...]
    mu1, var1 = _bn_stats(s1_ref[...], m1_ref[...], w1, b1)
    mu2, var2 = _bn_stats(s2_ref[...], m2_ref[...], w2, b2)
    x = jnp.dot(he_ref[...], w1, preferred_element_type=jnp.float32) + b1
    xn = (x - mu1) * lax.rsqrt(var1 + _BN_EPS) * g1_ref[...] + be1_ref[...]
    hr = jnp.maximum(xn, 0.0)
    x2 = jnp.dot(hr, w2, preferred_element_type=jnp.float32) + b2
    xn2 = (x2 - mu2) * lax.rsqrt(var2 + _BN_EPS) * g_ref[...] + be_ref[...]
    he = _leaky(xn2)
    out_ref[...] = jnp.dot(he, wo2_ref[...],
                           preferred_element_type=jnp.float32) + bo2_ref[...]


def _tc_edge_out(he_in, s1, m1, s2, m2, p):
    full = lambda s: pl.BlockSpec(s, lambda i: tuple(0 for _ in s))
    return pl.pallas_call(
        _edge_out_body,
        grid=(_NBLK,),
        in_specs=[
            pl.BlockSpec((_BE, _D_EDGE), lambda i: (i, 0)),
            full((1, _D_EDGE)), full((_D_EDGE, _D_EDGE)),
            full((1, _HID)), full((_HID, _HID)),
            full((_D_EDGE, _HID)), full((1, _HID)), full((1, _HID)),
            full((1, _HID)),
            full((_HID, _HID)), full((1, _HID)), full((1, _HID)),
            full((1, _HID)),
            full((_HID, _OUT)), full((1, _OUT)),
        ],
        out_specs=pl.BlockSpec((_BE, _OUT), lambda i: (i, 0)),
        out_shape=jax.ShapeDtypeStruct((_E, _OUT), jnp.float32),
    )(he_in, s1, m1, s2, m2,
      p['ec_w1'], p['ec_b1'].reshape(1, -1), p['ec_g1'].reshape(1, -1),
      p['ec_be1'].reshape(1, -1),
      p['ec_w2'], p['ec_b2'].reshape(1, -1), p['ec_g'].reshape(1, -1),
      p['ec_be'].reshape(1, -1),
      p['wo2'], p['bo2'].reshape(1, -1))


def kernel(h, edge_index, efeat, params):
    p = params
    src2d = edge_index[0].reshape(_E // _CHUNK, _CHUNK)
    dst2d = edge_index[1].reshape(_E // _CHUNK, _CHUNK)
    nc_eps = p['nc_eps'].reshape(1, 1)
    ec_eps = p['ec_eps'].reshape(1, 1)

    (h_src,) = _sc_gather(h.astype(jnp.bfloat16), [src2d], _D_IN,
                          jnp.bfloat16)
    msg = _tc_msg(h_src, efeat, p['A'].astype(jnp.bfloat16),
                  p['Bmat'].astype(jnp.bfloat16), nc_eps)
    agg2 = _sc_scatter_add(msg, dst2d)
    v_emb, hp = _tc_node(agg2, p)
    hp_src, hp_dst = _sc_gather(hp, [src2d, dst2d], _D_EDGE, jnp.bfloat16)
    he_in, s1, m1 = _tc_hein(efeat, hp_src, hp_dst, ec_eps)
    s2, m2 = _tc_stats2(he_in, s1, m1, p)
    e_emb = _tc_edge_out(he_in, s1, m1, s2, m2, p)
    return (v_emb, e_emb)


# speedup vs baseline: 1.3469x; 1.3469x over previous
import functools

import jax
import jax.numpy as jnp
from jax import lax
from jax.experimental import pallas as pl
from jax.experimental.pallas import tpu as pltpu
from jax.experimental.pallas import tpu_sc as plsc

_N = 10000
_E = 160000
_D_IN = 128
_D_EDGE = 16
_HID = 64
_OUT = 64
_BN_EPS = 1e-5

_CHUNK = 128
_NCHUNKS = _E // _CHUNK
_NW = 32
_CHUNKS_PER_W = -(-_NCHUNKS // _NW)
_NPAD = 10240
_ROWS_PER_SUB = _NPAD // 16

_BE = 3200
_NBLK = _E // _BE


def _sc_mesh():
    return plsc.VectorSubcoreMesh(core_axis_name="c", subcore_axis_name="s")


def _worker_id():
    return lax.axis_index("s") * 2 + lax.axis_index("c")


_IO_ROWS = 640
_IO_SUB = _IO_ROWS // _CHUNK
_N_IO = _E // _IO_ROWS
_K_PER_W = -(-_N_IO // _NW)


def _make_gather_body(njobs, width, dtype, io_rows):
    io_sub = io_rows // _CHUNK
    n_io = _E // io_rows
    k_per_w = -(-n_io // _NW)
    assert k_per_w % 2 == 0

    def body(*refs):
        idxs = refs[:njobs]
        table = refs[njobs]
        outs = refs[njobs + 1:2 * njobs + 1]
        sc = refs[2 * njobs + 1:]
        wid = _worker_id()

        def guard(k):
            return (k >= 0) & ((wid + k * _NW) < n_io)

        def addr(k):
            c = wid + k * _NW
            return (pl.multiple_of(c * io_sub, 8),
                    pl.multiple_of(c * io_rows, 8))

        def fire(k, b):
            @pl.when(guard(k))
            def _():
                i0, _r0 = addr(k)
                for j in range(njobs):
                    idx_v = sc[8 * j + b]
                    rows_v = sc[8 * j + 2 + b]
                    gsem = sc[8 * j + 4 + b]
                    pltpu.sync_copy(idxs[j].at[pl.ds(i0, io_sub)], idx_v)
                    for s in range(io_sub):
                        pltpu.async_copy(
                            table.at[idx_v.at[s]],
                            rows_v.at[pl.ds(s * _CHUNK, _CHUNK)], gsem)

        def drain_gather_fire_wb(k, b):
            @pl.when(guard(k))
            def _():
                _i0, r0 = addr(k)
                for j in range(njobs):
                    idx_v = sc[8 * j + b]
                    rows_v = sc[8 * j + 2 + b]
                    gsem = sc[8 * j + 4 + b]
                    wsem = sc[8 * j + 6 + b]
                    for s in range(io_sub):
                        pltpu.make_async_copy(
                            table.at[idx_v.at[s]],
                            rows_v.at[pl.ds(s * _CHUNK, _CHUNK)], gsem).wait()
                    pltpu.async_copy(rows_v, outs[j].at[pl.ds(r0, io_rows)],
                                     wsem)

        def drain_wb(k, b):
            @pl.when(guard(k))
            def _():
                _i0, r0 = addr(k)
                for j in range(njobs):
                    rows_v = sc[8 * j + 2 + b]
                    wsem = sc[8 * j + 6 + b]
                    pltpu.make_async_copy(
                        rows_v, outs[j].at[pl.ds(r0, io_rows)], wsem).wait()

        def step(i, carry):
            kk = 2 * i
            drain_wb(kk - 2, 0)
            fire(kk, 0)
            drain_wb(kk - 1, 1)
            fire(kk + 1, 1)
            drain_gather_fire_wb(kk, 0)
            drain_gather_fire_wb(kk + 1, 1)
            return carry

        lax.fori_loop(0, k_per_w // 2, step, 0)
        drain_wb(k_per_w - 2, 0)
        drain_wb(k_per_w - 1, 1)

    return body


def _sc_gather(table, idxs, width, dtype, io_rows):
    njobs = len(idxs)
    io_sub = io_rows // _CHUNK
    scratch = []
    for _ in range(njobs):
        scratch += [pltpu.VMEM((io_sub, _CHUNK), jnp.int32)] * 2
        scratch += [pltpu.VMEM((io_rows, width), dtype)] * 2
        scratch += [pltpu.SemaphoreType.DMA] * 4
    out = pl.kernel(
        _make_gather_body(njobs, width, dtype, io_rows),
        out_type=[jax.ShapeDtypeStruct((_E, width), dtype)] * njobs,
        mesh=_sc_mesh(),
        scratch_types=scratch,
        compiler_params=pltpu.CompilerParams(use_tc_tiling_on_sc=False),
    )(*idxs, table)
    return out


def _scatter_body(dst_hbm, msg_hbm, zeros_hbm, out_hbm,
                  idx0, idx1, msg0, msg1, msem0, msem1, asem0, asem1, agg_sh):
    cid = lax.axis_index("c")
    sid = lax.axis_index("s")
    wid = _worker_id()
    idx_b = (idx0, idx1)
    msg_b = (msg0, msg1)
    msem_b = (msem0, msem1)
    asem_b = (asem0, asem1)

    row0 = pl.multiple_of(sid * _ROWS_PER_SUB, 8)
    pltpu.sync_copy(zeros_hbm, agg_sh.at[pl.ds(row0, _ROWS_PER_SUB)])
    plsc.subcore_barrier()

    def guard(k):
        return (k >= 0) & ((wid + k * _NW) < _N_IO)

    def addr(k):
        c = wid + k * _NW
        return (pl.multiple_of(c * _IO_SUB, 8),
                pl.multiple_of(c * _IO_ROWS, 8))

    def fire(k, b):
        @pl.when(guard(k))
        def _():
            i0, r0 = addr(k)
            pltpu.sync_copy(dst_hbm.at[pl.ds(i0, _IO_SUB)], idx_b[b])
            pltpu.async_copy(msg_hbm.at[pl.ds(r0, _IO_ROWS)], msg_b[b],
                             msem_b[b])

    def drain_msg_fire_add(k, b):
        @pl.when(guard(k))
        def _():
            _i0, r0 = addr(k)
            pltpu.make_async_copy(msg_hbm.at[pl.ds(r0, _IO_ROWS)], msg_b[b],
                                  msem_b[b]).wait()
            for s in range(_IO_SUB):
                pltpu.async_copy(msg_b[b].at[pl.ds(s * _CHUNK, _CHUNK)],
                                 agg_sh.at[idx_b[b].at[s]], asem_b[b],
                                 add=True)

    def drain_add(k, b):
        @pl.when(guard(k))
        def _():
            for s in range(_IO_SUB):
                pltpu.make_async_copy(msg_b[b].at[pl.ds(s * _CHUNK, _CHUNK)],
                                      agg_sh.at[idx_b[b].at[s]],
                                      asem_b[b]).wait()

    def step(i, carry):
        kk = 2 * i
        drain_add(kk - 2, 0)
        fire(kk, 0)
        drain_add(kk - 1, 1)
        fire(kk + 1, 1)
        drain_msg_fire_add(kk, 0)
        drain_msg_fire_add(kk + 1, 1)
        return carry

    lax.fori_loop(0, _K_PER_W // 2, step, 0)
    drain_add(_K_PER_W - 2, 0)
    drain_add(_K_PER_W - 1, 1)
    plsc.subcore_barrier()
    out0 = pl.multiple_of(cid * _NPAD + row0, 8)
    pltpu.sync_copy(agg_sh.at[pl.ds(row0, _ROWS_PER_SUB)],
                    out_hbm.at[pl.ds(out0, _ROWS_PER_SUB)])


def _sc_scatter_add(msg, dst2d):
    zeros = jnp.zeros((_ROWS_PER_SUB, _HID), jnp.float32)
    return pl.kernel(
        _scatter_body,
        out_type=jax.ShapeDtypeStruct((2 * _NPAD, _HID), jnp.float32),
        mesh=_sc_mesh(),
        scratch_types=[
            pltpu.VMEM((_IO_SUB, _CHUNK), jnp.int32),
            pltpu.VMEM((_IO_SUB, _CHUNK), jnp.int32),
            pltpu.VMEM((_IO_ROWS, _HID), jnp.float32),
            pltpu.VMEM((_IO_ROWS, _HID), jnp.float32),
            pltpu.SemaphoreType.DMA,
            pltpu.SemaphoreType.DMA,
            pltpu.SemaphoreType.DMA,
            pltpu.SemaphoreType.DMA,
            pltpu.VMEM_SHARED((_NPAD, _HID), jnp.float32),
        ],
        compiler_params=pltpu.CompilerParams(use_tc_tiling_on_sc=False),
    )(dst2d, msg, zeros)


_BEP = _BE // 8


def _msg_body(hs_ref, ef_ref, a_ref, b_ref, eps_ref, out_ref):
    hs = hs_ref[...].astype(jnp.bfloat16)
    ef = ef_ref[...]
    acc = jnp.dot(hs, b_ref[...], preferred_element_type=jnp.float32)
    for f in range(_D_EDGE):
        acc += ef[:, f:f + 1] * jnp.dot(hs, a_ref[f],
                                        preferred_element_type=jnp.float32)
    out_ref[...] = (1.0 + eps_ref[0, 0]) * acc


def _tc_msg(h_src, efeat, a3, bmat, eps):
    return pl.pallas_call(
        _msg_body,
        grid=(_NBLK,),
        in_specs=[
            pl.BlockSpec((_BE, _D_IN), lambda i: (i, 0)),
            pl.BlockSpec((_BE, _D_EDGE), lambda i: (i, 0)),
            pl.BlockSpec((_D_EDGE, _D_IN, _HID), lambda i: (0, 0, 0)),
            pl.BlockSpec((_D_IN, _HID), lambda i: (0, 0)),
            pl.BlockSpec((1, 1), lambda i: (0, 0)),
        ],
        out_specs=pl.BlockSpec((_BE, _HID), lambda i: (i, 0)),
        out_shape=jax.ShapeDtypeStruct((_E, _HID), jnp.float32),
    )(h_src, efeat, a3, bmat, eps)


def _bn_cols(x, g, b):
    mu = jnp.mean(x, axis=0, keepdims=True)
    xc = x - mu
    var = jnp.mean(xc * xc, axis=0, keepdims=True)
    return xc * lax.rsqrt(var + _BN_EPS) * g + b


def _leaky(x):
    return jnp.where(x >= 0, x, 0.01 * x)


def _node_body(agg_ref, w1_ref, b1_ref, g1_ref, be1_ref, w2_ref, b2_ref,
               g_ref, be_ref, wo1_ref, bo1_ref, pw_ref, pb_ref,
               v_ref, hp_ref):
    agg = agg_ref[0:_N, :] + agg_ref[_NPAD:_NPAD + _N, :]
    x = jnp.dot(agg, w1_ref[...], preferred_element_type=jnp.float32) + b1_ref[...]
    hr = jnp.maximum(_bn_cols(x, g1_ref[...], be1_ref[...]), 0.0)
    x2 = jnp.dot(hr, w2_ref[...], preferred_element_type=jnp.float32) + b2_ref[...]
    h1 = _leaky(_bn_cols(x2, g_ref[...], be_ref[...]))
    v_ref[...] = jnp.dot(h1, wo1_ref[...], preferred_element_type=jnp.float32) + bo1_ref[...]
    hp_ref[...] = jnp.dot(h1, pw_ref[...], preferred_element_type=jnp.float32) + pb_ref[...]


def _tc_node(agg2, p):
    full = lambda s: pl.BlockSpec(s, lambda: tuple(0 for _ in s))
    return pl.pallas_call(
        _node_body,
        in_specs=[
            full((2 * _NPAD, _HID)),
            full((_HID, _HID)), full((1, _HID)), full((1, _HID)), full((1, _HID)),
            full((_HID, _HID)), full((1, _HID)), full((1, _HID)), full((1, _HID)),
            full((_HID, _OUT)), full((1, _OUT)),
            full((_HID, _D_EDGE)), full((1, _D_EDGE)),
        ],
        out_specs=[full((_N, _OUT)), full((_N, _D_EDGE))],
        out_shape=[
            jax.ShapeDtypeStruct((_N, _OUT), jnp.float32),
            jax.ShapeDtypeStruct((_N, _D_EDGE), jnp.float32),
        ],
    )(agg2,
      p['nc_w1'], p['nc_b1'].reshape(1, -1), p['nc_g1'].reshape(1, -1),
      p['nc_be1'].reshape(1, -1),
      p['nc_w2'], p['nc_b2'].reshape(1, -1), p['nc_g'].reshape(1, -1),
      p['nc_be'].reshape(1, -1),
      p['wo1'], p['bo1'].reshape(1, -1),
      p['ec_pw'], p['ec_pb'].reshape(1, -1))


def _hein_body(efp_ref, sp_ref, dp_ref, eps_ref, out_ref, s1_ref, m1_ref):
    i = pl.program_id(0)
    he = (1.0 + eps_ref[0, 0]) * efp_ref[...] + sp_ref[...] + dp_ref[...]
    out_ref[...] = he
    s = jnp.sum(he, axis=0, keepdims=True)
    m = lax.dot_general(he, he, (((0,), (0,)), ((), ())),
                        preferred_element_type=jnp.float32)

    @pl.when(i == 0)
    def _():
        s1_ref[...] = s
        m1_ref[...] = m

    @pl.when(i > 0)
    def _():
        s1_ref[...] += s
        m1_ref[...] += m


def _tc_hein(efp, hp_srcp, hp_dstp, eps):
    blk = pl.BlockSpec((_BEP, 128), lambda i: (i, 0))
    return pl.pallas_call(
        _hein_body,
        grid=(_NBLK,),
        in_specs=[blk, blk, blk, pl.BlockSpec((1, 1), lambda i: (0, 0))],
        out_specs=[
            pl.BlockSpec((_BEP, 128), lambda i: (i, 0)),
            pl.BlockSpec((1, 128), lambda i: (0, 0)),
            pl.BlockSpec((128, 128), lambda i: (0, 0)),
        ],
        out_shape=[
            jax.ShapeDtypeStruct((_E // 8, 128), jnp.float32),
            jax.ShapeDtypeStruct((1, 128), jnp.float32),
            jax.ShapeDtypeStruct((128, 128), jnp.float32),
        ],
    )(efp, hp_srcp, hp_dstp, eps)


def _unpack_stats1(s1p, m1p):
    s1 = s1p[:, 0:_D_EDGE]
    m1 = m1p[0:_D_EDGE, 0:_D_EDGE]
    for b in range(1, 8):
        s1 = s1 + s1p[:, b * _D_EDGE:(b + 1) * _D_EDGE]
        m1 = m1 + m1p[b * _D_EDGE:(b + 1) * _D_EDGE,
                      b * _D_EDGE:(b + 1) * _D_EDGE]
    return s1, m1


def _bn_stats(s, m, w, b):
    mean_y = s / _E
    mw = jnp.dot(mean_y, w, preferred_element_type=jnp.float32)
    mu = mw + b
    diag = jnp.sum(w * jnp.dot(m, w, preferred_element_type=jnp.float32),
                   axis=0, keepdims=True)
    ex2 = diag / _E + 2.0 * b * mw + b * b
    return mu, ex2 - mu * mu


def _tile8(v):
    return jnp.concatenate([v] * 8, axis=1)


def _fold8(v, w):
    acc = v[:, 0:w]
    for b in range(1, 8):
        acc = acc + v[:, b * w:(b + 1) * w]
    return acc


def _stats2_body(he_ref, s1_ref, m1_ref, w1big_ref, w1_ref, b1_ref,
                 g1_ref, be1_ref, s2_ref, m2_ref):
    i = pl.program_id(0)
    s1, m1 = _unpack_stats1(s1_ref[...], m1_ref[...])
    mu1, var1 = _bn_stats(s1, m1, w1_ref[...], b1_ref[...])
    scale = lax.rsqrt(var1 + _BN_EPS) * g1_ref[...]
    shift = _tile8(be1_ref[...] + (b1_ref[...] - mu1) * scale)
    hep = he_ref[...].astype(jnp.bfloat16)
    xp = jnp.dot(hep, w1big_ref[...], preferred_element_type=jnp.float32)
    hr = jnp.maximum(xp * _tile8(scale) + shift, 0.0)
    s = jnp.sum(hr, axis=0, keepdims=True)
    gp = lax.dot_general(hr, hr, (((0,), (0,)), ((), ())),
                         preferred_element_type=jnp.float32)
    m = gp[0:_HID, 0:_HID]
    for b in range(1, 8):
        m = m + gp[b * _HID:(b + 1) * _HID, b * _HID:(b + 1) * _HID]

    @pl.when(i == 0)
    def _():
        s2_ref[...] = _fold8(s, _HID)
        m2_ref[...] = m

    @pl.when(i > 0)
    def _():
        s2_ref[...] += _fold8(s, _HID)
        m2_ref[...] += m


def _tc_stats2(he_in, s1p, m1p, w1big, p):
    full = lambda s: pl.BlockSpec(s, lambda i: tuple(0 for _ in s))
    return pl.pallas_call(
        _stats2_body,
        grid=(_NBLK,),
        in_specs=[
            pl.BlockSpec((_BEP, 128), lambda i: (i, 0)),
            full((1, 128)), full((128, 128)),
            full((128, 8 * _HID)),
            full((_D_EDGE, _HID)), full((1, _HID)), full((1, _HID)),
            full((1, _HID)),
        ],
        out_specs=[full((1, _HID)), full((_HID, _HID))],
        out_shape=[
            jax.ShapeDtypeStruct((1, _HID), jnp.float32),
            jax.ShapeDtypeStruct((_HID, _HID), jnp.float32),
        ],
    )(he_in, s1p, m1p, w1big, p['ec_w1'], p['ec_b1'].reshape(1, -1),
      p['ec_g1'].reshape(1, -1), p['ec_be1'].reshape(1, -1))


def _edge_out_body(he_ref, s1_ref, m1_ref, s2_ref, m2_ref, w1big_ref,
                   w2big_ref, wo2big_ref, w1_ref, b1_ref, g1_ref, be1_ref,
                   w2_ref, b2_ref, g_ref, be_ref, bo2_ref, out_ref):
    s1, m1 = _unpack_stats1(s1_ref[...], m1_ref[...])
    mu1, var1 = _bn_stats(s1, m1, w1_ref[...], b1_ref[...])
    mu2, var2 = _bn_stats(s2_ref[...], m2_ref[...], w2_ref[...], b2_ref[...])
    scale1 = lax.rsqrt(var1 + _BN_EPS) * g1_ref[...]
    shift1 = _tile8(be1_ref[...] + (b1_ref[...] - mu1) * scale1)
    scale2 = lax.rsqrt(var2 + _BN_EPS) * g_ref[...]
    shift2 = _tile8(be_ref[...] + (b2_ref[...] - mu2) * scale2)
    hep = he_ref[...].astype(jnp.bfloat16)
    xp = jnp.dot(hep, w1big_ref[...], preferred_element_type=jnp.float32)
    hr = jnp.maximum(xp * _tile8(scale1) + shift1, 0.0)
    x2 = jnp.dot(hr.astype(jnp.bfloat16), w2big_ref[...],
                 preferred_element_type=jnp.float32)
    he = _leaky(x2 * _tile8(scale2) + shift2)
    out_ref[...] = (jnp.dot(he.astype(jnp.bfloat16), wo2big_ref[...],
                            preferred_element_type=jnp.float32)
                    + _tile8(bo2_ref[...]))


def _tc_edge_out(he_in, s1p, m1p, s2, m2, w1big, w2big, wo2big, p):
    full = lambda s: pl.BlockSpec(s, lambda i: tuple(0 for _ in s))
    return pl.pallas_call(
        _edge_out_body,
        grid=(_NBLK,),
        in_specs=[
            pl.BlockSpec((_BEP, 128), lambda i: (i, 0)),
            full((1, 128)), full((128, 128)),
            full((1, _HID)), full((_HID, _HID)),
            full((128, 8 * _HID)), full((8 * _HID, 8 * _HID)),
            full((8 * _HID, 8 * _OUT)),
            full((_D_EDGE, _HID)), full((1, _HID)), full((1, _HID)),
            full((1, _HID)),
            full((_HID, _HID)), full((1, _HID)), full((1, _HID)),
            full((1, _HID)),
            full((1, _OUT)),
        ],
        out_specs=pl.BlockSpec((_BEP, 8 * _OUT), lambda i: (i, 0)),
        out_shape=jax.ShapeDtypeStruct((_E // 8, 8 * _OUT), jnp.float32),
    )(he_in, s1p, m1p, s2, m2, w1big, w2big, wo2big,
      p['ec_w1'], p['ec_b1'].reshape(1, -1), p['ec_g1'].reshape(1, -1),
      p['ec_be1'].reshape(1, -1),
      p['ec_w2'], p['ec_b2'].reshape(1, -1), p['ec_g'].reshape(1, -1),
      p['ec_be'].reshape(1, -1),
      p['bo2'].reshape(1, -1))


def kernel(h, edge_index, efeat, params):
    p = params
    src2d = edge_index[0].reshape(_E // _CHUNK, _CHUNK)
    dst2d = edge_index[1].reshape(_E // _CHUNK, _CHUNK)
    nc_eps = p['nc_eps'].reshape(1, 1)
    ec_eps = p['ec_eps'].reshape(1, 1)
    efp = efeat.reshape(_E // 8, 128)
    a3 = p['A'].astype(jnp.bfloat16)
    eye8 = jnp.eye(8, dtype=jnp.float32)
    w1big = jnp.kron(eye8, p['ec_w1']).astype(jnp.bfloat16)
    w2big = jnp.kron(eye8, p['ec_w2']).astype(jnp.bfloat16)
    wo2big = jnp.kron(eye8, p['wo2']).astype(jnp.bfloat16)

    (h_src,) = _sc_gather(h, [src2d], _D_IN, jnp.float32, 256)
    msg = _tc_msg(h_src, efeat, a3, p['Bmat'].astype(jnp.bfloat16), nc_eps)
    agg2 = _sc_scatter_add(msg, dst2d)
    v_emb, hp = _tc_node(agg2, p)
    hp_src, hp_dst = _sc_gather(hp, [src2d, dst2d], _D_EDGE, jnp.float32,
                                _IO_ROWS)
    he_in, s1p, m1p = _tc_hein(efp,
                               hp_src.reshape(_E // 8, 128),
                               hp_dst.reshape(_E // 8, 128), ec_eps)
    s2, m2 = _tc_stats2(he_in, s1p, m1p, w1big, p)
    e_emb_p = _tc_edge_out(he_in, s1p, m1p, s2, m2, w1big, w2big, wo2big, p)
    return (v_emb, e_emb_p.reshape(_E, _OUT))


# speedup vs baseline: 1.4473x; 1.0745x over previous
import functools

import jax
import jax.numpy as jnp
from jax import lax
from jax.experimental import pallas as pl
from jax.experimental.pallas import tpu as pltpu
from jax.experimental.pallas import tpu_sc as plsc

_N = 10000
_E = 160000
_D_IN = 128
_D_EDGE = 16
_HID = 64
_OUT = 64
_BN_EPS = 1e-5

_CHUNK = 128
_NCHUNKS = _E // _CHUNK
_NW = 32
_CHUNKS_PER_W = -(-_NCHUNKS // _NW)
_NPAD = 10240
_ROWS_PER_SUB = _NPAD // 16

_BE = 3200
_NBLK = _E // _BE


def _sc_mesh():
    return plsc.VectorSubcoreMesh(core_axis_name="c", subcore_axis_name="s")


def _worker_id():
    return lax.axis_index("s") * 2 + lax.axis_index("c")


_IO_ROWS = 640
_IO_SUB = _IO_ROWS // _CHUNK
_N_IO = _E // _IO_ROWS
_K_PER_W = -(-_N_IO // _NW)


def _make_gather_body(njobs, width, dtype, io_rows):
    io_sub = io_rows // _CHUNK
    n_io = _E // io_rows
    k_per_w = -(-n_io // _NW)
    assert k_per_w % 2 == 0

    def body(*refs):
        idxs = refs[:njobs]
        table = refs[njobs]
        outs = refs[njobs + 1:2 * njobs + 1]
        sc = refs[2 * njobs + 1:]
        wid = _worker_id()

        def guard(k):
            return (k >= 0) & ((wid + k * _NW) < n_io)

        def addr(k):
            c = wid + k * _NW
            return (pl.multiple_of(c * io_sub, 8),
                    pl.multiple_of(c * io_rows, 8))

        def fire(k, b):
            @pl.when(guard(k))
            def _():
                i0, _r0 = addr(k)
                for j in range(njobs):
                    idx_v = sc[8 * j + b]
                    rows_v = sc[8 * j + 2 + b]
                    gsem = sc[8 * j + 4 + b]
                    pltpu.sync_copy(idxs[j].at[pl.ds(i0, io_sub)], idx_v)
                    for s in range(io_sub):
                        pltpu.async_copy(
                            table.at[idx_v.at[s]],
                            rows_v.at[pl.ds(s * _CHUNK, _CHUNK)], gsem)

        def drain_gather_fire_wb(k, b):
            @pl.when(guard(k))
            def _():
                _i0, r0 = addr(k)
                for j in range(njobs):
                    idx_v = sc[8 * j + b]
                    rows_v = sc[8 * j + 2 + b]
                    gsem = sc[8 * j + 4 + b]
                    wsem = sc[8 * j + 6 + b]
                    for s in range(io_sub):
                        pltpu.make_async_copy(
                            table.at[idx_v.at[s]],
                            rows_v.at[pl.ds(s * _CHUNK, _CHUNK)], gsem).wait()
                    pltpu.async_copy(rows_v, outs[j].at[pl.ds(r0, io_rows)],
                                     wsem)

        def drain_wb(k, b):
            @pl.when(guard(k))
            def _():
                _i0, r0 = addr(k)
                for j in range(njobs):
                    rows_v = sc[8 * j + 2 + b]
                    wsem = sc[8 * j + 6 + b]
                    pltpu.make_async_copy(
                        rows_v, outs[j].at[pl.ds(r0, io_rows)], wsem).wait()

        def step(i, carry):
            kk = 2 * i
            drain_wb(kk - 2, 0)
            fire(kk, 0)
            drain_wb(kk - 1, 1)
            fire(kk + 1, 1)
            drain_gather_fire_wb(kk, 0)
            drain_gather_fire_wb(kk + 1, 1)
            return carry

        lax.fori_loop(0, k_per_w // 2, step, 0)
        drain_wb(k_per_w - 2, 0)
        drain_wb(k_per_w - 1, 1)

    return body


def _sc_gather(table, idxs, width, dtype, io_rows):
    njobs = len(idxs)
    io_sub = io_rows // _CHUNK
    scratch = []
    for _ in range(njobs):
        scratch += [pltpu.VMEM((io_sub, _CHUNK), jnp.int32)] * 2
        scratch += [pltpu.VMEM((io_rows, width), dtype)] * 2
        scratch += [pltpu.SemaphoreType.DMA] * 4
    out = pl.kernel(
        _make_gather_body(njobs, width, dtype, io_rows),
        out_type=[jax.ShapeDtypeStruct((_E, width), dtype)] * njobs,
        mesh=_sc_mesh(),
        scratch_types=scratch,
        compiler_params=pltpu.CompilerParams(use_tc_tiling_on_sc=False),
    )(*idxs, table)
    return out


def _make_scatter_body(n_edges, io_rows, edge_off):
    io_sub = io_rows // _CHUNK
    n_io = n_edges // io_rows
    k_per_w = -(-n_io // _NW)
    assert k_per_w % 2 == 0
    off_sub = edge_off // _CHUNK

    def body(dst_hbm, msg_hbm, zeros_hbm, out_hbm,
             idx0, idx1, msg0, msg1, msem0, msem1, asem0, asem1, agg_sh):
        cid = lax.axis_index("c")
        sid = lax.axis_index("s")
        wid = _worker_id()
        idx_b = (idx0, idx1)
        msg_b = (msg0, msg1)
        msem_b = (msem0, msem1)
        asem_b = (asem0, asem1)

        row0 = pl.multiple_of(sid * _ROWS_PER_SUB, 8)
        pltpu.sync_copy(zeros_hbm, agg_sh.at[pl.ds(row0, _ROWS_PER_SUB)])
        plsc.subcore_barrier()

        def guard(k):
            return (k >= 0) & ((wid + k * _NW) < n_io)

        def addr(k):
            c = wid + k * _NW
            return (off_sub + c * io_sub,
                    pl.multiple_of(c * io_rows, 8))

        def fire(k, b):
            @pl.when(guard(k))
            def _():
                i0, r0 = addr(k)
                pltpu.sync_copy(dst_hbm.at[pl.ds(i0, io_sub)], idx_b[b])
                pltpu.async_copy(msg_hbm.at[pl.ds(r0, io_rows)], msg_b[b],
                                 msem_b[b])

        def drain_msg_fire_add(k, b):
            @pl.when(guard(k))
            def _():
                _i0, r0 = addr(k)
                pltpu.make_async_copy(msg_hbm.at[pl.ds(r0, io_rows)],
                                      msg_b[b], msem_b[b]).wait()
                for s in range(io_sub):
                    pltpu.async_copy(msg_b[b].at[pl.ds(s * _CHUNK, _CHUNK)],
                                     agg_sh.at[idx_b[b].at[s]], asem_b[b],
                                     add=True)

        def drain_add(k, b):
            @pl.when(guard(k))
            def _():
                for s in range(io_sub):
                    pltpu.make_async_copy(
                        msg_b[b].at[pl.ds(s * _CHUNK, _CHUNK)],
                        agg_sh.at[idx_b[b].at[s]], asem_b[b]).wait()

        def step(i, carry):
            kk = 2 * i
            drain_add(kk - 2, 0)
            fire(kk, 0)
            drain_add(kk - 1, 1)
            fire(kk + 1, 1)
            drain_msg_fire_add(kk, 0)
            drain_msg_fire_add(kk + 1, 1)
            return carry

        lax.fori_loop(0, k_per_w // 2, step, 0)
        drain_add(k_per_w - 2, 0)
        drain_add(k_per_w - 1, 1)
        plsc.subcore_barrier()
        out0 = pl.multiple_of(cid * _NPAD + row0, 8)
        pltpu.sync_copy(agg_sh.at[pl.ds(row0, _ROWS_PER_SUB)],
                        out_hbm.at[pl.ds(out0, _ROWS_PER_SUB)])

    return body


def _sc_scatter_add(msg, dst2d, edge_off, io_rows=128):
    n_edges = msg.shape[0]
    io_sub = io_rows // _CHUNK
    zeros = jnp.zeros((_ROWS_PER_SUB, 128), jnp.float32)
    return pl.kernel(
        _make_scatter_body(n_edges, io_rows, edge_off),
        out_type=jax.ShapeDtypeStruct((2 * _NPAD, 128), jnp.float32),
        mesh=_sc_mesh(),
        scratch_types=[
            pltpu.VMEM((io_sub, _CHUNK), jnp.int32),
            pltpu.VMEM((io_sub, _CHUNK), jnp.int32),
            pltpu.VMEM((io_rows, 128), jnp.float32),
            pltpu.VMEM((io_rows, 128), jnp.float32),
            pltpu.SemaphoreType.DMA,
            pltpu.SemaphoreType.DMA,
            pltpu.SemaphoreType.DMA,
            pltpu.SemaphoreType.DMA,
            pltpu.VMEM_SHARED((_NPAD, 128), jnp.float32),
        ],
        compiler_params=pltpu.CompilerParams(use_tc_tiling_on_sc=False),
    )(dst2d, msg, zeros)


_BEP = _BE // 8


def _msg_body(hs_ref, ef_ref, a_ref, b_ref, eps_ref, out_ref):
    hs = hs_ref[...].astype(jnp.bfloat16)
    ef = ef_ref[...]
    acc = jnp.dot(hs, b_ref[...], preferred_element_type=jnp.float32)
    for f in range(_D_EDGE):
        acc += ef[:, f:f + 1] * jnp.dot(hs, a_ref[f],
                                        preferred_element_type=jnp.float32)
    acc = (1.0 + eps_ref[0, 0]) * acc
    out_ref[...] = jnp.concatenate([acc, jnp.zeros_like(acc)], axis=1)


def _tc_msg(h_src, efeat, a3, bmat, eps, blk_off, nblk):
    return pl.pallas_call(
        _msg_body,
        grid=(nblk,),
        in_specs=[
            pl.BlockSpec((_BE, _D_IN), lambda i: (i + blk_off, 0)),
            pl.BlockSpec((_BE, _D_EDGE), lambda i: (i + blk_off, 0)),
            pl.BlockSpec((_D_EDGE, _D_IN, _HID), lambda i: (0, 0, 0)),
            pl.BlockSpec((_D_IN, _HID), lambda i: (0, 0)),
            pl.BlockSpec((1, 1), lambda i: (0, 0)),
        ],
        out_specs=pl.BlockSpec((_BE, 2 * _HID), lambda i: (i, 0)),
        out_shape=jax.ShapeDtypeStruct((nblk * _BE, 2 * _HID), jnp.float32),
    )(h_src, efeat, a3, bmat, eps)


def _bn_cols(x, g, b):
    mu = jnp.mean(x, axis=0, keepdims=True)
    xc = x - mu
    var = jnp.mean(xc * xc, axis=0, keepdims=True)
    return xc * lax.rsqrt(var + _BN_EPS) * g + b


def _leaky(x):
    return jnp.where(x >= 0, x, 0.01 * x)


def _node_body(agga_ref, aggb_ref, w1_ref, b1_ref, g1_ref, be1_ref,
               w2_ref, b2_ref, g_ref, be_ref, wo1_ref, bo1_ref, pw_ref,
               pb_ref, v_ref, hp_ref):
    agg = (agga_ref[0:_N, 0:_HID] + agga_ref[_NPAD:_NPAD + _N, 0:_HID]
           + aggb_ref[0:_N, 0:_HID] + aggb_ref[_NPAD:_NPAD + _N, 0:_HID])
    x = jnp.dot(agg, w1_ref[...], preferred_element_type=jnp.float32) + b1_ref[...]
    hr = jnp.maximum(_bn_cols(x, g1_ref[...], be1_ref[...]), 0.0)
    x2 = jnp.dot(hr, w2_ref[...], preferred_element_type=jnp.float32) + b2_ref[...]
    h1 = _leaky(_bn_cols(x2, g_ref[...], be_ref[...]))
    v_ref[...] = jnp.dot(h1, wo1_ref[...], preferred_element_type=jnp.float32) + bo1_ref[...]
    hp_ref[...] = jnp.dot(h1, pw_ref[...], preferred_element_type=jnp.float32) + pb_ref[...]


def _tc_node(agga, aggb, p):
    full = lambda s: pl.BlockSpec(s, lambda: tuple(0 for _ in s))
    return pl.pallas_call(
        _node_body,
        in_specs=[
            full((2 * _NPAD, 128)), full((2 * _NPAD, 128)),
            full((_HID, _HID)), full((1, _HID)), full((1, _HID)), full((1, _HID)),
            full((_HID, _HID)), full((1, _HID)), full((1, _HID)), full((1, _HID)),
            full((_HID, _OUT)), full((1, _OUT)),
            full((_HID, _D_EDGE)), full((1, _D_EDGE)),
        ],
        out_specs=[full((_N, _OUT)), full((_N, _D_EDGE))],
        out_shape=[
            jax.ShapeDtypeStruct((_N, _OUT), jnp.float32),
            jax.ShapeDtypeStruct((_N, _D_EDGE), jnp.float32),
        ],
    )(agga, aggb,
      p['nc_w1'], p['nc_b1'].reshape(1, -1), p['nc_g1'].reshape(1, -1),
      p['nc_be1'].reshape(1, -1),
      p['nc_w2'], p['nc_b2'].reshape(1, -1), p['nc_g'].reshape(1, -1),
      p['nc_be'].reshape(1, -1),
      p['wo1'], p['bo1'].reshape(1, -1),
      p['ec_pw'], p['ec_pb'].reshape(1, -1))


def _hein_body(efp_ref, sp_ref, dp_ref, eps_ref, out_ref, s1_ref, m1_ref):
    i = pl.program_id(0)
    he = (1.0 + eps_ref[0, 0]) * efp_ref[...] + sp_ref[...] + dp_ref[...]
    out_ref[...] = he
    s = jnp.sum(he, axis=0, keepdims=True)
    m = lax.dot_general(he, he, (((0,), (0,)), ((), ())),
                        preferred_element_type=jnp.float32)

    @pl.when(i == 0)
    def _():
        s1_ref[...] = s
        m1_ref[...] = m

    @pl.when(i > 0)
    def _():
        s1_ref[...] += s
        m1_ref[...] += m


def _tc_hein(efp, hp_srcp, hp_dstp, eps):
    blk = pl.BlockSpec((_BEP, 128), lambda i: (i, 0))
    return pl.pallas_call(
        _hein_body,
        grid=(_NBLK,),
        in_specs=[blk, blk, blk, pl.BlockSpec((1, 1), lambda i: (0, 0))],
        out_specs=[
            pl.BlockSpec((_BEP, 128), lambda i: (i, 0)),
            pl.BlockSpec((1, 128), lambda i: (0, 0)),
            pl.BlockSpec((128, 128), lambda i: (0, 0)),
        ],
        out_shape=[
            jax.ShapeDtypeStruct((_E // 8, 128), jnp.float32),
            jax.ShapeDtypeStruct((1, 128), jnp.float32),
            jax.ShapeDtypeStruct((128, 128), jnp.float32),
        ],
    )(efp, hp_srcp, hp_dstp, eps)


def _unpack_stats1(s1p, m1p):
    s1 = s1p[:, 0:_D_EDGE]
    m1 = m1p[0:_D_EDGE, 0:_D_EDGE]
    for b in range(1, 8):
        s1 = s1 + s1p[:, b * _D_EDGE:(b + 1) * _D_EDGE]
        m1 = m1 + m1p[b * _D_EDGE:(b + 1) * _D_EDGE,
                      b * _D_EDGE:(b + 1) * _D_EDGE]
    return s1, m1


def _bn_stats(s, m, w, b):
    mean_y = s / _E
    mw = jnp.dot(mean_y, w, preferred_element_type=jnp.float32)
    mu = mw + b
    diag = jnp.sum(w * jnp.dot(m, w, preferred_element_type=jnp.float32),
                   axis=0, keepdims=True)
    ex2 = diag / _E + 2.0 * b * mw + b * b
    return mu, ex2 - mu * mu


def _tile8(v):
    return jnp.concatenate([v] * 8, axis=1)


def _fold8(v, w):
    acc = v[:, 0:w]
    for b in range(1, 8):
        acc = acc + v[:, b * w:(b + 1) * w]
    return acc


def _stats2_body(he_ref, s1_ref, m1_ref, w1big_ref, w1_ref, b1_ref,
                 g1_ref, be1_ref, s2_ref, m2_ref):
    i = pl.program_id(0)
    s1, m1 = _unpack_stats1(s1_ref[...], m1_ref[...])
    mu1, var1 = _bn_stats(s1, m1, w1_ref[...], b1_ref[...])
    scale = lax.rsqrt(var1 + _BN_EPS) * g1_ref[...]
    shift = _tile8(be1_ref[...] + (b1_ref[...] - mu1) * scale)
    hep = he_ref[...].astype(jnp.bfloat16)
    xp = jnp.dot(hep, w1big_ref[...], preferred_element_type=jnp.float32)
    hr = jnp.maximum(xp * _tile8(scale) + shift, 0.0)
    s = jnp.sum(hr, axis=0, keepdims=True)
    gp = lax.dot_general(hr, hr, (((0,), (0,)), ((), ())),
                         preferred_element_type=jnp.float32)
    m = gp[0:_HID, 0:_HID]
    for b in range(1, 8):
        m = m + gp[b * _HID:(b + 1) * _HID, b * _HID:(b + 1) * _HID]

    @pl.when(i == 0)
    def _():
        s2_ref[...] = _fold8(s, _HID)
        m2_ref[...] = m

    @pl.when(i > 0)
    def _():
        s2_ref[...] += _fold8(s, _HID)
        m2_ref[...] += m


def _tc_stats2(he_in, s1p, m1p, w1big, p):
    full = lambda s: pl.BlockSpec(s, lambda i: tuple(0 for _ in s))
    return pl.pallas_call(
        _stats2_body,
        grid=(_NBLK,),
        in_specs=[
            pl.BlockSpec((_BEP, 128), lambda i: (i, 0)),
            full((1, 128)), full((128, 128)),
            full((128, 8 * _HID)),
            full((_D_EDGE, _HID)), full((1, _HID)), full((1, _HID)),
            full((1, _HID)),
        ],
        out_specs=[full((1, _HID)), full((_HID, _HID))],
        out_shape=[
            jax.ShapeDtypeStruct((1, _HID), jnp.float32),
            jax.ShapeDtypeStruct((_HID, _HID), jnp.float32),
        ],
    )(he_in, s1p, m1p, w1big, p['ec_w1'], p['ec_b1'].reshape(1, -1),
      p['ec_g1'].reshape(1, -1), p['ec_be1'].reshape(1, -1))


def _edge_out_body(he_ref, s1_ref, m1_ref, s2_ref, m2_ref, w1big_ref,
                   w2big_ref, wo2big_ref, w1_ref, b1_ref, g1_ref, be1_ref,
                   w2_ref, b2_ref, g_ref, be_ref, bo2_ref, out_ref):
    s1, m1 = _unpack_stats1(s1_ref[...], m1_ref[...])
    mu1, var1 = _bn_stats(s1, m1, w1_ref[...], b1_ref[...])
    mu2, var2 = _bn_stats(s2_ref[...], m2_ref[...], w2_ref[...], b2_ref[...])
    scale1 = lax.rsqrt(var1 + _BN_EPS) * g1_ref[...]
    shift1 = _tile8(be1_ref[...] + (b1_ref[...] - mu1) * scale1)
    scale2 = lax.rsqrt(var2 + _BN_EPS) * g_ref[...]
    shift2 = _tile8(be_ref[...] + (b2_ref[...] - mu2) * scale2)
    hep = he_ref[...].astype(jnp.bfloat16)
    xp = jnp.dot(hep, w1big_ref[...], preferred_element_type=jnp.float32)
    hr = jnp.maximum(xp * _tile8(scale1) + shift1, 0.0)
    x2 = jnp.dot(hr.astype(jnp.bfloat16), w2big_ref[...],
                 preferred_element_type=jnp.float32)
    he = _leaky(x2 * _tile8(scale2) + shift2)
    out_ref[...] = (jnp.dot(he.astype(jnp.bfloat16), wo2big_ref[...],
                            preferred_element_type=jnp.float32)
                    + _tile8(bo2_ref[...]))


def _tc_edge_out(he_in, s1p, m1p, s2, m2, w1big, w2big, wo2big, p):
    full = lambda s: pl.BlockSpec(s, lambda i: tuple(0 for _ in s))
    return pl.pallas_call(
        _edge_out_body,
        grid=(_NBLK,),
        in_specs=[
            pl.BlockSpec((_BEP, 128), lambda i: (i, 0)),
            full((1, 128)), full((128, 128)),
            full((1, _HID)), full((_HID, _HID)),
            full((128, 8 * _HID)), full((8 * _HID, 8 * _HID)),
            full((8 * _HID, 8 * _OUT)),
            full((_D_EDGE, _HID)), full((1, _HID)), full((1, _HID)),
            full((1, _HID)),
            full((_HID, _HID)), full((1, _HID)), full((1, _HID)),
            full((1, _HID)),
            full((1, _OUT)),
        ],
        out_specs=pl.BlockSpec((_BEP, 8 * _OUT), lambda i: (i, 0)),
        out_shape=jax.ShapeDtypeStruct((_E // 8, 8 * _OUT), jnp.float32),
    )(he_in, s1p, m1p, s2, m2, w1big, w2big, wo2big,
      p['ec_w1'], p['ec_b1'].reshape(1, -1), p['ec_g1'].reshape(1, -1),
      p['ec_be1'].reshape(1, -1),
      p['ec_w2'], p['ec_b2'].reshape(1, -1), p['ec_g'].reshape(1, -1),
      p['ec_be'].reshape(1, -1),
      p['bo2'].reshape(1, -1))


def kernel(h, edge_index, efeat, params):
    p = params
    src2d = edge_index[0].reshape(_E // _CHUNK, _CHUNK)
    dst2d = edge_index[1].reshape(_E // _CHUNK, _CHUNK)
    nc_eps = p['nc_eps'].reshape(1, 1)
    ec_eps = p['ec_eps'].reshape(1, 1)
    efp = efeat.reshape(_E // 8, 128)
    a3 = p['A'].astype(jnp.bfloat16)
    eye8 = jnp.eye(8, dtype=jnp.float32)
    w1big = jnp.kron(eye8, p['ec_w1']).astype(jnp.bfloat16)
    w2big = jnp.kron(eye8, p['ec_w2']).astype(jnp.bfloat16)
    wo2big = jnp.kron(eye8, p['wo2']).astype(jnp.bfloat16)

    (h_src,) = _sc_gather(h, [src2d], _D_IN, jnp.float32, 256)
    bmat_bf = p['Bmat'].astype(jnp.bfloat16)
    nblk_a = 30
    msg_a = _tc_msg(h_src, efeat, a3, bmat_bf, nc_eps, 0, nblk_a)
    agg_a = _sc_scatter_add(msg_a, dst2d, 0)
    msg_b = _tc_msg(h_src, efeat, a3, bmat_bf, nc_eps, nblk_a,
                    _NBLK - nblk_a)
    agg_b = _sc_scatter_add(msg_b, dst2d, nblk_a * _BE)
    v_emb, hp = _tc_node(agg_a, agg_b, p)
    hp_src, hp_dst = _sc_gather(hp, [src2d, dst2d], _D_EDGE, jnp.float32,
                                _IO_ROWS)
    he_in, s1p, m1p = _tc_hein(efp,
                               hp_src.reshape(_E // 8, 128),
                               hp_dst.reshape(_E // 8, 128), ec_eps)
    s2, m2 = _tc_stats2(he_in, s1p, m1p, w1big, p)
    e_emb_p = _tc_edge_out(he_in, s1p, m1p, s2, m2, w1big, w2big, wo2big, p)
    return (v_emb, e_emb_p.reshape(_E, _OUT))


# speedup vs baseline: 1.4576x; 1.0071x over previous
import functools

import jax
import jax.numpy as jnp
from jax import lax
from jax.experimental import pallas as pl
from jax.experimental.pallas import tpu as pltpu
from jax.experimental.pallas import tpu_sc as plsc

_N = 10000
_E = 160000
_D_IN = 128
_D_EDGE = 16
_HID = 64
_OUT = 64
_BN_EPS = 1e-5

_CHUNK = 128
_NCHUNKS = _E // _CHUNK
_NW = 32
_CHUNKS_PER_W = -(-_NCHUNKS // _NW)
_NPAD = 10240
_ROWS_PER_SUB = _NPAD // 16

_BE = 3200
_NBLK = _E // _BE


def _sc_mesh():
    return plsc.VectorSubcoreMesh(core_axis_name="c", subcore_axis_name="s")


def _worker_id():
    return lax.axis_index("s") * 2 + lax.axis_index("c")


_IO_ROWS = 640
_IO_SUB = _IO_ROWS // _CHUNK
_N_IO = _E // _IO_ROWS
_K_PER_W = -(-_N_IO // _NW)


def _make_gather_body(njobs, width, dtype, io_rows):
    io_sub = io_rows // _CHUNK
    n_io = _E // io_rows
    k_per_w = -(-n_io // _NW)
    assert k_per_w % 2 == 0

    def body(*refs):
        idxs = refs[:njobs]
        table = refs[njobs]
        outs = refs[njobs + 1:2 * njobs + 1]
        sc = refs[2 * njobs + 1:]
        wid = _worker_id()

        def guard(k):
            return (k >= 0) & ((wid + k * _NW) < n_io)

        def addr(k):
            c = wid + k * _NW
            return (pl.multiple_of(c * io_sub, 8),
                    pl.multiple_of(c * io_rows, 8))

        def fire(k, b):
            @pl.when(guard(k))
            def _():
                i0, _r0 = addr(k)
                for j in range(njobs):
                    idx_v = sc[8 * j + b]
                    rows_v = sc[8 * j + 2 + b]
                    gsem = sc[8 * j + 4 + b]
                    pltpu.sync_copy(idxs[j].at[pl.ds(i0, io_sub)], idx_v)
                    for s in range(io_sub):
                        pltpu.async_copy(
                            table.at[idx_v.at[s]],
                            rows_v.at[pl.ds(s * _CHUNK, _CHUNK)], gsem)

        def drain_gather_fire_wb(k, b):
            @pl.when(guard(k))
            def _():
                _i0, r0 = addr(k)
                for j in range(njobs):
                    idx_v = sc[8 * j + b]
                    rows_v = sc[8 * j + 2 + b]
                    gsem = sc[8 * j + 4 + b]
                    wsem = sc[8 * j + 6 + b]
                    for s in range(io_sub):
                        pltpu.make_async_copy(
                            table.at[idx_v.at[s]],
                            rows_v.at[pl.ds(s * _CHUNK, _CHUNK)], gsem).wait()
                    pltpu.async_copy(rows_v, outs[j].at[pl.ds(r0, io_rows)],
                                     wsem)

        def drain_wb(k, b):
            @pl.when(guard(k))
            def _():
                _i0, r0 = addr(k)
                for j in range(njobs):
                    rows_v = sc[8 * j + 2 + b]
                    wsem = sc[8 * j + 6 + b]
                    pltpu.make_async_copy(
                        rows_v, outs[j].at[pl.ds(r0, io_rows)], wsem).wait()

        def step(i, carry):
            kk = 2 * i
            drain_wb(kk - 2, 0)
            fire(kk, 0)
            drain_wb(kk - 1, 1)
            fire(kk + 1, 1)
            drain_gather_fire_wb(kk, 0)
            drain_gather_fire_wb(kk + 1, 1)
            return carry

        lax.fori_loop(0, k_per_w // 2, step, 0)
        drain_wb(k_per_w - 2, 0)
        drain_wb(k_per_w - 1, 1)

    return body


def _sc_gather(table, idxs, width, dtype, io_rows):
    njobs = len(idxs)
    io_sub = io_rows // _CHUNK
    scratch = []
    for _ in range(njobs):
        scratch += [pltpu.VMEM((io_sub, _CHUNK), jnp.int32)] * 2
        scratch += [pltpu.VMEM((io_rows, width), dtype)] * 2
        scratch += [pltpu.SemaphoreType.DMA] * 4
    out = pl.kernel(
        _make_gather_body(njobs, width, dtype, io_rows),
        out_type=[jax.ShapeDtypeStruct((_E, width), dtype)] * njobs,
        mesh=_sc_mesh(),
        scratch_types=scratch,
        compiler_params=pltpu.CompilerParams(use_tc_tiling_on_sc=False),
    )(*idxs, table)
    return out


def _make_scatter_body(n_edges, io_rows, edge_off):
    io_sub = io_rows // _CHUNK
    n_io = n_edges // io_rows
    k_per_w = -(-n_io // _NW)
    assert k_per_w % 2 == 0
    off_sub = edge_off // _CHUNK

    def body(dst_hbm, msg_hbm, zeros_hbm, out_hbm,
             idx0, idx1, msg0, msg1, msem0, msem1, asem0, asem1, agg_sh):
        cid = lax.axis_index("c")
        sid = lax.axis_index("s")
        wid = _worker_id()
        idx_b = (idx0, idx1)
        msg_b = (msg0, msg1)
        msem_b = (msem0, msem1)
        asem_b = (asem0, asem1)

        row0 = pl.multiple_of(sid * _ROWS_PER_SUB, 8)
        pltpu.sync_copy(zeros_hbm, agg_sh.at[pl.ds(row0, _ROWS_PER_SUB)])
        plsc.subcore_barrier()

        def guard(k):
            return (k >= 0) & ((wid + k * _NW) < n_io)

        def addr(k):
            c = wid + k * _NW
            return (off_sub + c * io_sub,
                    pl.multiple_of(c * io_rows, 8))

        def fire(k, b):
            @pl.when(guard(k))
            def _():
                i0, r0 = addr(k)
                pltpu.sync_copy(dst_hbm.at[pl.ds(i0, io_sub)], idx_b[b])
                pltpu.async_copy(msg_hbm.at[pl.ds(r0, io_rows)], msg_b[b],
                                 msem_b[b])

        def drain_msg_fire_add(k, b):
            @pl.when(guard(k))
            def _():
                _i0, r0 = addr(k)
                pltpu.make_async_copy(msg_hbm.at[pl.ds(r0, io_rows)],
                                      msg_b[b], msem_b[b]).wait()
                for s in range(io_sub):
                    pltpu.async_copy(msg_b[b].at[pl.ds(s * _CHUNK, _CHUNK)],
                                     agg_sh.at[idx_b[b].at[s]], asem_b[b],
                                     add=True)

        def drain_add(k, b):
            @pl.when(guard(k))
            def _():
                for s in range(io_sub):
                    pltpu.make_async_copy(
                        msg_b[b].at[pl.ds(s * _CHUNK, _CHUNK)],
                        agg_sh.at[idx_b[b].at[s]], asem_b[b]).wait()

        def step(i, carry):
            kk = 2 * i
            drain_add(kk - 2, 0)
            fire(kk, 0)
            drain_add(kk - 1, 1)
            fire(kk + 1, 1)
            drain_msg_fire_add(kk, 0)
            drain_msg_fire_add(kk + 1, 1)
            return carry

        lax.fori_loop(0, k_per_w // 2, step, 0)
        drain_add(k_per_w - 2, 0)
        drain_add(k_per_w - 1, 1)
        plsc.subcore_barrier()
        out0 = pl.multiple_of(cid * _NPAD + row0, 8)
        pltpu.sync_copy(agg_sh.at[pl.ds(row0, _ROWS_PER_SUB)],
                        out_hbm.at[pl.ds(out0, _ROWS_PER_SUB)])

    return body


def _sc_scatter_add(msg, dst2d, edge_off, io_rows=128):
    n_edges = msg.shape[0]
    io_sub = io_rows // _CHUNK
    zeros = jnp.zeros((_ROWS_PER_SUB, 128), jnp.float32)
    return pl.kernel(
        _make_scatter_body(n_edges, io_rows, edge_off),
        out_type=jax.ShapeDtypeStruct((2 * _NPAD, 128), jnp.float32),
        mesh=_sc_mesh(),
        scratch_types=[
            pltpu.VMEM((io_sub, _CHUNK), jnp.int32),
            pltpu.VMEM((io_sub, _CHUNK), jnp.int32),
            pltpu.VMEM((io_rows, 128), jnp.float32),
            pltpu.VMEM((io_rows, 128), jnp.float32),
            pltpu.SemaphoreType.DMA,
            pltpu.SemaphoreType.DMA,
            pltpu.SemaphoreType.DMA,
            pltpu.SemaphoreType.DMA,
            pltpu.VMEM_SHARED((_NPAD, 128), jnp.float32),
        ],
        compiler_params=pltpu.CompilerParams(use_tc_tiling_on_sc=False),
    )(dst2d, msg, zeros)


_BEP = _BE // 8


def _msg_body(hs_ref, ef_ref, a_ref, b_ref, eps_ref, out_ref):
    hs = hs_ref[...].astype(jnp.bfloat16)
    ef = ef_ref[...]
    acc = jnp.dot(hs, b_ref[...], preferred_element_type=jnp.float32)
    for f in range(_D_EDGE):
        acc += ef[:, f:f + 1] * jnp.dot(hs, a_ref[f],
                                        preferred_element_type=jnp.float32)
    acc = (1.0 + eps_ref[0, 0]) * acc
    out_ref[...] = jnp.concatenate([acc, jnp.zeros_like(acc)], axis=1)


def _tc_msg(h_src, efeat, a3, bmat, eps, blk_off, nblk):
    return pl.pallas_call(
        _msg_body,
        grid=(nblk,),
        in_specs=[
            pl.BlockSpec((_BE, _D_IN), lambda i: (i + blk_off, 0)),
            pl.BlockSpec((_BE, _D_EDGE), lambda i: (i + blk_off, 0)),
            pl.BlockSpec((_D_EDGE, _D_IN, _HID), lambda i: (0, 0, 0)),
            pl.BlockSpec((_D_IN, _HID), lambda i: (0, 0)),
            pl.BlockSpec((1, 1), lambda i: (0, 0)),
        ],
        out_specs=pl.BlockSpec((_BE, 2 * _HID), lambda i: (i, 0)),
        out_shape=jax.ShapeDtypeStruct((nblk * _BE, 2 * _HID), jnp.float32),
    )(h_src, efeat, a3, bmat, eps)


def _bn_cols(x, g, b):
    mu = jnp.mean(x, axis=0, keepdims=True)
    xc = x - mu
    var = jnp.mean(xc * xc, axis=0, keepdims=True)
    return xc * lax.rsqrt(var + _BN_EPS) * g + b


def _leaky(x):
    return jnp.where(x >= 0, x, 0.01 * x)


def _node_body(agga_ref, aggb_ref, w1_ref, b1_ref, g1_ref, be1_ref,
               w2_ref, b2_ref, g_ref, be_ref, wo1_ref, bo1_ref, pw_ref,
               pb_ref, v_ref, hp_ref):
    agg = (agga_ref[0:_N, 0:_HID] + agga_ref[_NPAD:_NPAD + _N, 0:_HID]
           + aggb_ref[0:_N, 0:_HID] + aggb_ref[_NPAD:_NPAD + _N, 0:_HID])
    x = jnp.dot(agg, w1_ref[...], preferred_element_type=jnp.float32) + b1_ref[...]
    hr = jnp.maximum(_bn_cols(x, g1_ref[...], be1_ref[...]), 0.0)
    x2 = jnp.dot(hr, w2_ref[...], preferred_element_type=jnp.float32) + b2_ref[...]
    h1 = _leaky(_bn_cols(x2, g_ref[...], be_ref[...]))
    v_ref[...] = jnp.dot(h1, wo1_ref[...], preferred_element_type=jnp.float32) + bo1_ref[...]
    hp_ref[...] = jnp.dot(h1, pw_ref[...], preferred_element_type=jnp.float32) + pb_ref[...]


def _tc_node(agga, aggb, p):
    full = lambda s: pl.BlockSpec(s, lambda: tuple(0 for _ in s))
    return pl.pallas_call(
        _node_body,
        in_specs=[
            full((2 * _NPAD, 128)), full((2 * _NPAD, 128)),
            full((_HID, _HID)), full((1, _HID)), full((1, _HID)), full((1, _HID)),
            full((_HID, _HID)), full((1, _HID)), full((1, _HID)), full((1, _HID)),
            full((_HID, _OUT)), full((1, _OUT)),
            full((_HID, _D_EDGE)), full((1, _D_EDGE)),
        ],
        out_specs=[full((_N, _OUT)), full((_N, _D_EDGE))],
        out_shape=[
            jax.ShapeDtypeStruct((_N, _OUT), jnp.float32),
            jax.ShapeDtypeStruct((_N, _D_EDGE), jnp.float32),
        ],
    )(agga, aggb,
      p['nc_w1'], p['nc_b1'].reshape(1, -1), p['nc_g1'].reshape(1, -1),
      p['nc_be1'].reshape(1, -1),
      p['nc_w2'], p['nc_b2'].reshape(1, -1), p['nc_g'].reshape(1, -1),
      p['nc_be'].reshape(1, -1),
      p['wo1'], p['bo1'].reshape(1, -1),
      p['ec_pw'], p['ec_pb'].reshape(1, -1))


def _hein_body(efp_ref, sp_ref, dp_ref, eps_ref, out_ref, s1_ref, m1_ref):
    i = pl.program_id(0)
    he = (1.0 + eps_ref[0, 0]) * efp_ref[...] + sp_ref[...] + dp_ref[...]
    out_ref[...] = he
    s = jnp.sum(he, axis=0, keepdims=True)
    m = lax.dot_general(he, he, (((0,), (0,)), ((), ())),
                        preferred_element_type=jnp.float32)

    @pl.when(i == 0)
    def _():
        s1_ref[...] = s
        m1_ref[...] = m

    @pl.when(i > 0)
    def _():
        s1_ref[...] += s
        m1_ref[...] += m


def _tc_hein(efp, hp_srcp, hp_dstp, eps):
    blk = pl.BlockSpec((_BEP, 128), lambda i: (i, 0))
    return pl.pallas_call(
        _hein_body,
        grid=(_NBLK,),
        in_specs=[blk, blk, blk, pl.BlockSpec((1, 1), lambda i: (0, 0))],
        out_specs=[
            pl.BlockSpec((_BEP, 128), lambda i: (i, 0)),
            pl.BlockSpec((1, 128), lambda i: (0, 0)),
            pl.BlockSpec((128, 128), lambda i: (0, 0)),
        ],
        out_shape=[
            jax.ShapeDtypeStruct((_E // 8, 128), jnp.float32),
            jax.ShapeDtypeStruct((1, 128), jnp.float32),
            jax.ShapeDtypeStruct((128, 128), jnp.float32),
        ],
    )(efp, hp_srcp, hp_dstp, eps)


def _unpack_stats1(s1p, m1p):
    s1 = s1p[:, 0:_D_EDGE]
    m1 = m1p[0:_D_EDGE, 0:_D_EDGE]
    for b in range(1, 8):
        s1 = s1 + s1p[:, b * _D_EDGE:(b + 1) * _D_EDGE]
        m1 = m1 + m1p[b * _D_EDGE:(b + 1) * _D_EDGE,
                      b * _D_EDGE:(b + 1) * _D_EDGE]
    return s1, m1


def _bn_stats(s, m, w, b):
    mean_y = s / _E
    mw = jnp.dot(mean_y, w, preferred_element_type=jnp.float32)
    mu = mw + b
    diag = jnp.sum(w * jnp.dot(m, w, preferred_element_type=jnp.float32),
                   axis=0, keepdims=True)
    ex2 = diag / _E + 2.0 * b * mw + b * b
    return mu, ex2 - mu * mu


def _tile8(v):
    return jnp.concatenate([v] * 8, axis=1)


def _fold8(v, w):
    acc = v[:, 0:w]
    for b in range(1, 8):
        acc = acc + v[:, b * w:(b + 1) * w]
    return acc


def _stats2_body(he_ref, s1_ref, m1_ref, w1big_ref, w1_ref, b1_ref,
                 g1_ref, be1_ref, s2_ref, m2_ref):
    i = pl.program_id(0)
    s1, m1 = _unpack_stats1(s1_ref[...], m1_ref[...])
    mu1, var1 = _bn_stats(s1, m1, w1_ref[...], b1_ref[...])
    scale = lax.rsqrt(var1 + _BN_EPS) * g1_ref[...]
    shift = _tile8(be1_ref[...] + (b1_ref[...] - mu1) * scale)
    hep = he_ref[...].astype(jnp.bfloat16)
    xp = jnp.dot(hep, w1big_ref[...], preferred_element_type=jnp.float32)
    hr = jnp.maximum(xp * _tile8(scale) + shift, 0.0)
    s = jnp.sum(hr, axis=0, keepdims=True)
    hrb = hr.astype(jnp.bfloat16)
    gp = lax.dot_general(hrb, hrb, (((0,), (0,)), ((), ())),
                         preferred_element_type=jnp.float32)
    m = gp[0:_HID, 0:_HID]
    for b in range(1, 8):
        m = m + gp[b * _HID:(b + 1) * _HID, b * _HID:(b + 1) * _HID]

    @pl.when(i == 0)
    def _():
        s2_ref[...] = _fold8(s, _HID)
        m2_ref[...] = m

    @pl.when(i > 0)
    def _():
        s2_ref[...] += _fold8(s, _HID)
        m2_ref[...] += m


def _tc_stats2(he_in, s1p, m1p, w1big, p):
    full = lambda s: pl.BlockSpec(s, lambda i: tuple(0 for _ in s))
    return pl.pallas_call(
        _stats2_body,
        grid=(_NBLK,),
        in_specs=[
            pl.BlockSpec((_BEP, 128), lambda i: (i, 0)),
            full((1, 128)), full((128, 128)),
            full((128, 8 * _HID)),
            full((_D_EDGE, _HID)), full((1, _HID)), full((1, _HID)),
            full((1, _HID)),
        ],
        out_specs=[full((1, _HID)), full((_HID, _HID))],
        out_shape=[
            jax.ShapeDtypeStruct((1, _HID), jnp.float32),
            jax.ShapeDtypeStruct((_HID, _HID), jnp.float32),
        ],
    )(he_in, s1p, m1p, w1big, p['ec_w1'], p['ec_b1'].reshape(1, -1),
      p['ec_g1'].reshape(1, -1), p['ec_be1'].reshape(1, -1))


def _edge_out_body(he_ref, s1_ref, m1_ref, s2_ref, m2_ref, w1big_ref,
                   w2big_ref, wo2big_ref, w1_ref, b1_ref, g1_ref, be1_ref,
                   w2_ref, b2_ref, g_ref, be_ref, bo2_ref, out_ref):
    s1, m1 = _unpack_stats1(s1_ref[...], m1_ref[...])
    mu1, var1 = _bn_stats(s1, m1, w1_ref[...], b1_ref[...])
    mu2, var2 = _bn_stats(s2_ref[...], m2_ref[...], w2_ref[...], b2_ref[...])
    scale1 = lax.rsqrt(var1 + _BN_EPS) * g1_ref[...]
    shift1 = _tile8(be1_ref[...] + (b1_ref[...] - mu1) * scale1)
    scale2 = lax.rsqrt(var2 + _BN_EPS) * g_ref[...]
    shift2 = _tile8(be_ref[...] + (b2_ref[...] - mu2) * scale2)
    hep = he_ref[...].astype(jnp.bfloat16)
    xp = jnp.dot(hep, w1big_ref[...], preferred_element_type=jnp.float32)
    hr = jnp.maximum(xp * _tile8(scale1) + shift1, 0.0)
    x2 = jnp.dot(hr.astype(jnp.bfloat16), w2big_ref[...],
                 preferred_element_type=jnp.float32)
    he = _leaky(x2 * _tile8(scale2) + shift2)
    out_ref[...] = (jnp.dot(he.astype(jnp.bfloat16), wo2big_ref[...],
                            preferred_element_type=jnp.float32)
                    + _tile8(bo2_ref[...]))


def _tc_edge_out(he_in, s1p, m1p, s2, m2, w1big, w2big, wo2big, p):
    full = lambda s: pl.BlockSpec(s, lambda i: tuple(0 for _ in s))
    return pl.pallas_call(
        _edge_out_body,
        grid=(_NBLK,),
        in_specs=[
            pl.BlockSpec((_BEP, 128), lambda i: (i, 0)),
            full((1, 128)), full((128, 128)),
            full((1, _HID)), full((_HID, _HID)),
            full((128, 8 * _HID)), full((8 * _HID, 8 * _HID)),
            full((8 * _HID, 8 * _OUT)),
            full((_D_EDGE, _HID)), full((1, _HID)), full((1, _HID)),
            full((1, _HID)),
            full((_HID, _HID)), full((1, _HID)), full((1, _HID)),
            full((1, _HID)),
            full((1, _OUT)),
        ],
        out_specs=pl.BlockSpec((_BEP, 8 * _OUT), lambda i: (i, 0)),
        out_shape=jax.ShapeDtypeStruct((_E // 8, 8 * _OUT), jnp.float32),
    )(he_in, s1p, m1p, s2, m2, w1big, w2big, wo2big,
      p['ec_w1'], p['ec_b1'].reshape(1, -1), p['ec_g1'].reshape(1, -1),
      p['ec_be1'].reshape(1, -1),
      p['ec_w2'], p['ec_b2'].reshape(1, -1), p['ec_g'].reshape(1, -1),
      p['ec_be'].reshape(1, -1),
      p['bo2'].reshape(1, -1))


def kernel(h, edge_index, efeat, params):
    p = params
    src2d = edge_index[0].reshape(_E // _CHUNK, _CHUNK)
    dst2d = edge_index[1].reshape(_E // _CHUNK, _CHUNK)
    nc_eps = p['nc_eps'].reshape(1, 1)
    ec_eps = p['ec_eps'].reshape(1, 1)
    efp = efeat.reshape(_E // 8, 128)
    a3 = p['A'].astype(jnp.bfloat16)
    eye8 = jnp.eye(8, dtype=jnp.float32)
    w1big = jnp.kron(eye8, p['ec_w1']).astype(jnp.bfloat16)
    w2big = jnp.kron(eye8, p['ec_w2']).astype(jnp.bfloat16)
    wo2big = jnp.kron(eye8, p['wo2']).astype(jnp.bfloat16)

    (h_src,) = _sc_gather(h, [src2d], _D_IN, jnp.float32, 256)
    bmat_bf = p['Bmat'].astype(jnp.bfloat16)
    nblk_a = 35
    msg_a = _tc_msg(h_src, efeat, a3, bmat_bf, nc_eps, 0, nblk_a)
    agg_a = _sc_scatter_add(msg_a, dst2d, 0)
    msg_b = _tc_msg(h_src, efeat, a3, bmat_bf, nc_eps, nblk_a,
                    _NBLK - nblk_a)
    agg_b = _sc_scatter_add(msg_b, dst2d, nblk_a * _BE)
    v_emb, hp = _tc_node(agg_a, agg_b, p)
    hp_src, hp_dst = _sc_gather(hp, [src2d, dst2d], _D_EDGE, jnp.float32,
                                _IO_ROWS)
    he_in, s1p, m1p = _tc_hein(efp,
                               hp_src.reshape(_E // 8, 128),
                               hp_dst.reshape(_E // 8, 128), ec_eps)
    s2, m2 = _tc_stats2(he_in, s1p, m1p, w1big, p)
    e_emb_p = _tc_edge_out(he_in, s1p, m1p, s2, m2, w1big, w2big, wo2big, p)
    return (v_emb, e_emb_p.reshape(_E, _OUT))


# speedup vs baseline: 1.4623x; 1.0032x over previous
import functools

import jax
import jax.numpy as jnp
from jax import lax
from jax.experimental import pallas as pl
from jax.experimental.pallas import tpu as pltpu
from jax.experimental.pallas import tpu_sc as plsc

_N = 10000
_E = 160000
_D_IN = 128
_D_EDGE = 16
_HID = 64
_OUT = 64
_BN_EPS = 1e-5

_CHUNK = 128
_NCHUNKS = _E // _CHUNK
_NW = 32
_CHUNKS_PER_W = -(-_NCHUNKS // _NW)
_NPAD = 10240
_ROWS_PER_SUB = _NPAD // 16

_BE = 3200
_NBLK = _E // _BE


def _sc_mesh():
    return plsc.VectorSubcoreMesh(core_axis_name="c", subcore_axis_name="s")


def _worker_id():
    return lax.axis_index("s") * 2 + lax.axis_index("c")


_IO_ROWS = 640
_IO_SUB = _IO_ROWS // _CHUNK
_N_IO = _E // _IO_ROWS
_K_PER_W = -(-_N_IO // _NW)


def _make_gather_body(njobs, width, dtype, io_rows, pack8):
    io_sub = io_rows // _CHUNK
    n_io = _E // io_rows
    k_per_w = -(-n_io // _NW)
    assert k_per_w % 2 == 0
    wb_rows = io_rows // 8 if pack8 else io_rows

    def body(*refs):
        idxs = refs[:njobs]
        table = refs[njobs]
        outs = refs[njobs + 1:2 * njobs + 1]
        sc = refs[2 * njobs + 1:]
        wid = _worker_id()

        def guard(k):
            return (k >= 0) & ((wid + k * _NW) < n_io)

        def addr(k):
            c = wid + k * _NW
            return (pl.multiple_of(c * io_sub, 8),
                    pl.multiple_of(c * wb_rows, 8))

        def wb_src(rows_v):
            return rows_v.reshape(wb_rows, 128) if pack8 else rows_v

        def fire(k, b):
            @pl.when(guard(k))
            def _():
                i0, _r0 = addr(k)
                for j in range(njobs):
                    idx_v = sc[8 * j + b]
                    rows_v = sc[8 * j + 2 + b]
                    gsem = sc[8 * j + 4 + b]
                    pltpu.sync_copy(idxs[j].at[pl.ds(i0, io_sub)], idx_v)
                    for s in range(io_sub):
                        pltpu.async_copy(
                            table.at[idx_v.at[s]],
                            rows_v.at[pl.ds(s * _CHUNK, _CHUNK)], gsem)

        def drain_gather_fire_wb(k, b):
            @pl.when(guard(k))
            def _():
                _i0, r0 = addr(k)
                for j in range(njobs):
                    idx_v = sc[8 * j + b]
                    rows_v = sc[8 * j + 2 + b]
                    gsem = sc[8 * j + 4 + b]
                    wsem = sc[8 * j + 6 + b]
                    for s in range(io_sub):
                        pltpu.make_async_copy(
                            table.at[idx_v.at[s]],
                            rows_v.at[pl.ds(s * _CHUNK, _CHUNK)], gsem).wait()
                    pltpu.async_copy(wb_src(rows_v),
                                     outs[j].at[pl.ds(r0, wb_rows)], wsem)

        def drain_wb(k, b):
            @pl.when(guard(k))
            def _():
                _i0, r0 = addr(k)
                for j in range(njobs):
                    rows_v = sc[8 * j + 2 + b]
                    wsem = sc[8 * j + 6 + b]
                    pltpu.make_async_copy(
                        wb_src(rows_v), outs[j].at[pl.ds(r0, wb_rows)],
                        wsem).wait()

        def step(i, carry):
            kk = 2 * i
            drain_wb(kk - 2, 0)
            fire(kk, 0)
            drain_wb(kk - 1, 1)
            fire(kk + 1, 1)
            drain_gather_fire_wb(kk, 0)
            drain_gather_fire_wb(kk + 1, 1)
            return carry

        lax.fori_loop(0, k_per_w // 2, step, 0)
        drain_wb(k_per_w - 2, 0)
        drain_wb(k_per_w - 1, 1)

    return body


def _sc_gather(table, idxs, width, dtype, io_rows, pack8=False):
    njobs = len(idxs)
    io_sub = io_rows // _CHUNK
    out_shape = ((_E // 8, 128) if pack8 else (_E, width))
    scratch = []
    for _ in range(njobs):
        scratch += [pltpu.VMEM((io_sub, _CHUNK), jnp.int32)] * 2
        scratch += [pltpu.VMEM((io_rows, width), dtype)] * 2
        scratch += [pltpu.SemaphoreType.DMA] * 4
    out = pl.kernel(
        _make_gather_body(njobs, width, dtype, io_rows, pack8),
        out_type=[jax.ShapeDtypeStruct(out_shape, dtype)] * njobs,
        mesh=_sc_mesh(),
        scratch_types=scratch,
        compiler_params=pltpu.CompilerParams(use_tc_tiling_on_sc=False),
    )(*idxs, table)
    return out


def _make_scatter_body(n_edges, io_rows, edge_off):
    io_sub = io_rows // _CHUNK
    n_io = n_edges // io_rows
    k_per_w = -(-n_io // _NW)
    assert k_per_w % 2 == 0
    off_sub = edge_off // _CHUNK

    def body(dst_hbm, msg_hbm, zeros_hbm, out_hbm,
             idx0, idx1, msg0, msg1, msem0, msem1, asem0, asem1, agg_sh):
        cid = lax.axis_index("c")
        sid = lax.axis_index("s")
        wid = _worker_id()
        idx_b = (idx0, idx1)
        msg_b = (msg0, msg1)
        msem_b = (msem0, msem1)
        asem_b = (asem0, asem1)

        row0 = pl.multiple_of(sid * _ROWS_PER_SUB, 8)
        pltpu.sync_copy(zeros_hbm, agg_sh.at[pl.ds(row0, _ROWS_PER_SUB)])
        plsc.subcore_barrier()

        def guard(k):
            return (k >= 0) & ((wid + k * _NW) < n_io)

        def addr(k):
            c = wid + k * _NW
            return (off_sub + c * io_sub,
                    pl.multiple_of(c * io_rows, 8))

        def fire(k, b):
            @pl.when(guard(k))
            def _():
                i0, r0 = addr(k)
                pltpu.sync_copy(dst_hbm.at[pl.ds(i0, io_sub)], idx_b[b])
                pltpu.async_copy(msg_hbm.at[pl.ds(r0, io_rows)], msg_b[b],
                                 msem_b[b])

        def drain_msg_fire_add(k, b):
            @pl.when(guard(k))
            def _():
                _i0, r0 = addr(k)
                pltpu.make_async_copy(msg_hbm.at[pl.ds(r0, io_rows)],
                                      msg_b[b], msem_b[b]).wait()
                for s in range(io_sub):
                    pltpu.async_copy(msg_b[b].at[pl.ds(s * _CHUNK, _CHUNK)],
                                     agg_sh.at[idx_b[b].at[s]], asem_b[b],
                                     add=True)

        def drain_add(k, b):
            @pl.when(guard(k))
            def _():
                for s in range(io_sub):
                    pltpu.make_async_copy(
                        msg_b[b].at[pl.ds(s * _CHUNK, _CHUNK)],
                        agg_sh.at[idx_b[b].at[s]], asem_b[b]).wait()

        def step(i, carry):
            kk = 2 * i
            drain_add(kk - 2, 0)
            fire(kk, 0)
            drain_add(kk - 1, 1)
            fire(kk + 1, 1)
            drain_msg_fire_add(kk, 0)
            drain_msg_fire_add(kk + 1, 1)
            return carry

        lax.fori_loop(0, k_per_w // 2, step, 0)
        drain_add(k_per_w - 2, 0)
        drain_add(k_per_w - 1, 1)
        plsc.subcore_barrier()
        out0 = pl.multiple_of(cid * _NPAD + row0, 8)
        pltpu.sync_copy(agg_sh.at[pl.ds(row0, _ROWS_PER_SUB)],
                        out_hbm.at[pl.ds(out0, _ROWS_PER_SUB)])

    return body


def _sc_scatter_add(msg, dst2d, edge_off, io_rows=128):
    n_edges = msg.shape[0]
    io_sub = io_rows // _CHUNK
    zeros = jnp.zeros((_ROWS_PER_SUB, 128), jnp.float32)
    return pl.kernel(
        _make_scatter_body(n_edges, io_rows, edge_off),
        out_type=jax.ShapeDtypeStruct((2 * _NPAD, 128), jnp.float32),
        mesh=_sc_mesh(),
        scratch_types=[
            pltpu.VMEM((io_sub, _CHUNK), jnp.int32),
            pltpu.VMEM((io_sub, _CHUNK), jnp.int32),
            pltpu.VMEM((io_rows, 128), jnp.float32),
            pltpu.VMEM((io_rows, 128), jnp.float32),
            pltpu.SemaphoreType.DMA,
            pltpu.SemaphoreType.DMA,
            pltpu.SemaphoreType.DMA,
            pltpu.SemaphoreType.DMA,
            pltpu.VMEM_SHARED((_NPAD, 128), jnp.float32),
        ],
        compiler_params=pltpu.CompilerParams(use_tc_tiling_on_sc=False),
    )(dst2d, msg, zeros)


_BEP = _BE // 8


def _msg_body(hs_ref, ef_ref, a_ref, b_ref, eps_ref, out_ref):
    hs = hs_ref[...].astype(jnp.bfloat16)
    ef = ef_ref[...]
    acc = jnp.dot(hs, b_ref[...], preferred_element_type=jnp.float32)
    for f in range(_D_EDGE):
        acc += ef[:, f:f + 1] * jnp.dot(hs, a_ref[f],
                                        preferred_element_type=jnp.float32)
    acc = (1.0 + eps_ref[0, 0]) * acc
    out_ref[...] = jnp.concatenate([acc, jnp.zeros_like(acc)], axis=1)


def _tc_msg(h_src, efeat, a4, bmat, eps, blk_off, nblk):
    return pl.pallas_call(
        _msg_body,
        grid=(nblk,),
        in_specs=[
            pl.BlockSpec((_BE, _D_IN), lambda i: (i + blk_off, 0)),
            pl.BlockSpec((_BE, _D_EDGE), lambda i: (i + blk_off, 0)),
            pl.BlockSpec((_D_EDGE, _D_IN, _HID), lambda i: (0, 0, 0)),
            pl.BlockSpec((_D_IN, _HID), lambda i: (0, 0)),
            pl.BlockSpec((1, 1), lambda i: (0, 0)),
        ],
        out_specs=pl.BlockSpec((_BE, 2 * _HID), lambda i: (i, 0)),
        out_shape=jax.ShapeDtypeStruct((nblk * _BE, 2 * _HID), jnp.float32),
    )(h_src, efeat, a4, bmat, eps)


def _bn_cols(x, g, b):
    mu = jnp.mean(x, axis=0, keepdims=True)
    xc = x - mu
    var = jnp.mean(xc * xc, axis=0, keepdims=True)
    return xc * lax.rsqrt(var + _BN_EPS) * g + b


def _leaky(x):
    return jnp.where(x >= 0, x, 0.01 * x)


def _node_body(agga_ref, aggb_ref, w1_ref, b1_ref, g1_ref, be1_ref,
               w2_ref, b2_ref, g_ref, be_ref, wo1_ref, bo1_ref, pw_ref,
               pb_ref, v_ref, hp_ref):
    agg = (agga_ref[0:_N, 0:_HID] + agga_ref[_NPAD:_NPAD + _N, 0:_HID]
           + aggb_ref[0:_N, 0:_HID] + aggb_ref[_NPAD:_NPAD + _N, 0:_HID])
    x = jnp.dot(agg, w1_ref[...], preferred_element_type=jnp.float32) + b1_ref[...]
    hr = jnp.maximum(_bn_cols(x, g1_ref[...], be1_ref[...]), 0.0)
    x2 = jnp.dot(hr, w2_ref[...], preferred_element_type=jnp.float32) + b2_ref[...]
    h1 = _leaky(_bn_cols(x2, g_ref[...], be_ref[...]))
    v_ref[...] = jnp.dot(h1, wo1_ref[...], preferred_element_type=jnp.float32) + bo1_ref[...]
    hp_ref[...] = jnp.dot(h1, pw_ref[...], preferred_element_type=jnp.float32) + pb_ref[...]


def _tc_node(agga, aggb, p):
    full = lambda s: pl.BlockSpec(s, lambda: tuple(0 for _ in s))
    return pl.pallas_call(
        _node_body,
        in_specs=[
            full((2 * _NPAD, 128)), full((2 * _NPAD, 128)),
            full((_HID, _HID)), full((1, _HID)), full((1, _HID)), full((1, _HID)),
            full((_HID, _HID)), full((1, _HID)), full((1, _HID)), full((1, _HID)),
            full((_HID, _OUT)), full((1, _OUT)),
            full((_HID, _D_EDGE)), full((1, _D_EDGE)),
        ],
        out_specs=[full((_N, _OUT)), full((_N, _D_EDGE))],
        out_shape=[
            jax.ShapeDtypeStruct((_N, _OUT), jnp.float32),
            jax.ShapeDtypeStruct((_N, _D_EDGE), jnp.float32),
        ],
    )(agga, aggb,
      p['nc_w1'], p['nc_b1'].reshape(1, -1), p['nc_g1'].reshape(1, -1),
      p['nc_be1'].reshape(1, -1),
      p['nc_w2'], p['nc_b2'].reshape(1, -1), p['nc_g'].reshape(1, -1),
      p['nc_be'].reshape(1, -1),
      p['wo1'], p['bo1'].reshape(1, -1),
      p['ec_pw'], p['ec_pb'].reshape(1, -1))


def _hein_body(efp_ref, sp_ref, dp_ref, eps_ref, out_ref, s1_ref, m1_ref):
    i = pl.program_id(0)
    he = (1.0 + eps_ref[0, 0]) * efp_ref[...] + sp_ref[...] + dp_ref[...]
    out_ref[...] = he
    s = jnp.sum(he, axis=0, keepdims=True)
    m = lax.dot_general(he, he, (((0,), (0,)), ((), ())),
                        preferred_element_type=jnp.float32)

    @pl.when(i == 0)
    def _():
        s1_ref[...] = s
        m1_ref[...] = m

    @pl.when(i > 0)
    def _():
        s1_ref[...] += s
        m1_ref[...] += m


def _tc_hein(efp, hp_srcp, hp_dstp, eps):
    blk = pl.BlockSpec((_BEP, 128), lambda i: (i, 0))
    return pl.pallas_call(
        _hein_body,
        grid=(_NBLK,),
        in_specs=[blk, blk, blk, pl.BlockSpec((1, 1), lambda i: (0, 0))],
        out_specs=[
            pl.BlockSpec((_BEP, 128), lambda i: (i, 0)),
            pl.BlockSpec((1, 128), lambda i: (0, 0)),
            pl.BlockSpec((128, 128), lambda i: (0, 0)),
        ],
        out_shape=[
            jax.ShapeDtypeStruct((_E // 8, 128), jnp.float32),
            jax.ShapeDtypeStruct((1, 128), jnp.float32),
            jax.ShapeDtypeStruct((128, 128), jnp.float32),
        ],
    )(efp, hp_srcp, hp_dstp, eps)


def _unpack_stats1(s1p, m1p):
    s1 = s1p[:, 0:_D_EDGE]
    m1 = m1p[0:_D_EDGE, 0:_D_EDGE]
    for b in range(1, 8):
        s1 = s1 + s1p[:, b * _D_EDGE:(b + 1) * _D_EDGE]
        m1 = m1 + m1p[b * _D_EDGE:(b + 1) * _D_EDGE,
                      b * _D_EDGE:(b + 1) * _D_EDGE]
    return s1, m1


def _bn_stats(s, m, w, b):
    mean_y = s / _E
    mw = jnp.dot(mean_y, w, preferred_element_type=jnp.float32)
    mu = mw + b
    diag = jnp.sum(w * jnp.dot(m, w, preferred_element_type=jnp.float32),
                   axis=0, keepdims=True)
    ex2 = diag / _E + 2.0 * b * mw + b * b
    return mu, ex2 - mu * mu


def _tile8(v):
    return jnp.concatenate([v] * 8, axis=1)


def _fold8(v, w):
    acc = v[:, 0:w]
    for b in range(1, 8):
        acc = acc + v[:, b * w:(b + 1) * w]
    return acc


def _stats2_body(he_ref, s1_ref, m1_ref, w1big_ref, w1_ref, b1_ref,
                 g1_ref, be1_ref, s2_ref, m2_ref):
    i = pl.program_id(0)
    s1, m1 = _unpack_stats1(s1_ref[...], m1_ref[...])
    mu1, var1 = _bn_stats(s1, m1, w1_ref[...], b1_ref[...])
    scale = lax.rsqrt(var1 + _BN_EPS) * g1_ref[...]
    shift = _tile8(be1_ref[...] + (b1_ref[...] - mu1) * scale)
    hep = he_ref[...].astype(jnp.bfloat16)
    xp = jnp.dot(hep, w1big_ref[...], preferred_element_type=jnp.float32)
    hr = jnp.maximum(xp * _tile8(scale) + shift, 0.0)
    s = jnp.sum(hr, axis=0, keepdims=True)
    hrb = hr.astype(jnp.bfloat16)
    gp = lax.dot_general(hrb, hrb, (((0,), (0,)), ((), ())),
                         preferred_element_type=jnp.float32)
    m = gp[0:_HID, 0:_HID]
    for b in range(1, 8):
        m = m + gp[b * _HID:(b + 1) * _HID, b * _HID:(b + 1) * _HID]

    @pl.when(i == 0)
    def _():
        s2_ref[...] = _fold8(s, _HID)
        m2_ref[...] = m

    @pl.when(i > 0)
    def _():
        s2_ref[...] += _fold8(s, _HID)
        m2_ref[...] += m


def _tc_stats2(he_in, s1p, m1p, w1big, p):
    full = lambda s: pl.BlockSpec(s, lambda i: tuple(0 for _ in s))
    return pl.pallas_call(
        _stats2_body,
        grid=(_NBLK,),
        in_specs=[
            pl.BlockSpec((_BEP, 128), lambda i: (i, 0)),
            full((1, 128)), full((128, 128)),
            full((128, 8 * _HID)),
            full((_D_EDGE, _HID)), full((1, _HID)), full((1, _HID)),
            full((1, _HID)),
        ],
        out_specs=[full((1, _HID)), full((_HID, _HID))],
        out_shape=[
            jax.ShapeDtypeStruct((1, _HID), jnp.float32),
            jax.ShapeDtypeStruct((_HID, _HID), jnp.float32),
        ],
    )(he_in, s1p, m1p, w1big, p['ec_w1'], p['ec_b1'].reshape(1, -1),
      p['ec_g1'].reshape(1, -1), p['ec_be1'].reshape(1, -1))


def _edge_out_body(he_ref, s1_ref, m1_ref, s2_ref, m2_ref, w1big_ref,
                   w2big_ref, wo2big_ref, w1_ref, b1_ref, g1_ref, be1_ref,
                   w2_ref, b2_ref, g_ref, be_ref, bo2_ref, out_ref):
    s1, m1 = _unpack_stats1(s1_ref[...], m1_ref[...])
    mu1, var1 = _bn_stats(s1, m1, w1_ref[...], b1_ref[...])
    mu2, var2 = _bn_stats(s2_ref[...], m2_ref[...], w2_ref[...], b2_ref[...])
    scale1 = lax.rsqrt(var1 + _BN_EPS) * g1_ref[...]
    shift1 = _tile8(be1_ref[...] + (b1_ref[...] - mu1) * scale1)
    scale2 = lax.rsqrt(var2 + _BN_EPS) * g_ref[...]
    shift2 = _tile8(be_ref[...] + (b2_ref[...] - mu2) * scale2)
    hep = he_ref[...].astype(jnp.bfloat16)
    xp = jnp.dot(hep, w1big_ref[...], preferred_element_type=jnp.float32)
    hr = jnp.maximum(xp * _tile8(scale1) + shift1, 0.0)
    x2 = jnp.dot(hr.astype(jnp.bfloat16), w2big_ref[...],
                 preferred_element_type=jnp.float32)
    he = _leaky(x2 * _tile8(scale2) + shift2)
    out_ref[...] = (jnp.dot(he.astype(jnp.bfloat16), wo2big_ref[...],
                            preferred_element_type=jnp.float32)
                    + _tile8(bo2_ref[...]))


def _tc_edge_out(he_in, s1p, m1p, s2, m2, w1big, w2big, wo2big, p):
    full = lambda s: pl.BlockSpec(s, lambda i: tuple(0 for _ in s))
    return pl.pallas_call(
        _edge_out_body,
        grid=(_NBLK,),
        in_specs=[
            pl.BlockSpec((_BEP, 128), lambda i: (i, 0)),
            full((1, 128)), full((128, 128)),
            full((1, _HID)), full((_HID, _HID)),
            full((128, 8 * _HID)), full((8 * _HID, 8 * _HID)),
            full((8 * _HID, 8 * _OUT)),
            full((_D_EDGE, _HID)), full((1, _HID)), full((1, _HID)),
            full((1, _HID)),
            full((_HID, _HID)), full((1, _HID)), full((1, _HID)),
            full((1, _HID)),
            full((1, _OUT)),
        ],
        out_specs=pl.BlockSpec((_BEP, 8 * _OUT), lambda i: (i, 0)),
        out_shape=jax.ShapeDtypeStruct((_E // 8, 8 * _OUT), jnp.float32),
    )(he_in, s1p, m1p, s2, m2, w1big, w2big, wo2big,
      p['ec_w1'], p['ec_b1'].reshape(1, -1), p['ec_g1'].reshape(1, -1),
      p['ec_be1'].reshape(1, -1),
      p['ec_w2'], p['ec_b2'].reshape(1, -1), p['ec_g'].reshape(1, -1),
      p['ec_be'].reshape(1, -1),
      p['bo2'].reshape(1, -1))


def kernel(h, edge_index, efeat, params):
    p = params
    src2d = edge_index[0].reshape(_E // _CHUNK, _CHUNK)
    dst2d = edge_index[1].reshape(_E // _CHUNK, _CHUNK)
    nc_eps = p['nc_eps'].reshape(1, 1)
    ec_eps = p['ec_eps'].reshape(1, 1)
    efp = efeat.reshape(_E // 8, 128)
    a4 = p['A'].astype(jnp.bfloat16)
    eye8 = jnp.eye(8, dtype=jnp.float32)
    w1big = jnp.kron(eye8, p['ec_w1']).astype(jnp.bfloat16)
    w2big = jnp.kron(eye8, p['ec_w2']).astype(jnp.bfloat16)
    wo2big = jnp.kron(eye8, p['wo2']).astype(jnp.bfloat16)

    (h_src,) = _sc_gather(h, [src2d], _D_IN, jnp.float32, 256)
    bmat_bf = p['Bmat'].astype(jnp.bfloat16)
    nblk_a = 35
    msg_a = _tc_msg(h_src, efeat, a4, bmat_bf, nc_eps, 0, nblk_a)
    agg_a = _sc_scatter_add(msg_a, dst2d, 0)
    msg_b = _tc_msg(h_src, efeat, a4, bmat_bf, nc_eps, nblk_a,
                    _NBLK - nblk_a)
    agg_b = _sc_scatter_add(msg_b, dst2d, nblk_a * _BE)
    v_emb, hp = _tc_node(agg_a, agg_b, p)
    hp_src, hp_dst = _sc_gather(hp, [src2d, dst2d], _D_EDGE, jnp.float32,
                                _IO_ROWS)
    he_in, s1p, m1p = _tc_hein(efp,
                               hp_src.reshape(_E // 8, 128),
                               hp_dst.reshape(_E // 8, 128), ec_eps)
    s2, m2 = _tc_stats2(he_in, s1p, m1p, w1big, p)
    e_emb_p = _tc_edge_out(he_in, s1p, m1p, s2, m2, w1big, w2big, wo2big, p)
    return (v_emb, e_emb_p.reshape(_E, _OUT))
